# Initial kernel scaffold; baseline (speedup 1.0000x reference)
import functools

import jax
import jax.numpy as jnp
from jax import lax
from jax.experimental import pallas as pl
from jax.experimental.pallas import tpu as pltpu
from jax.experimental.pallas import tpu_sc as plsc

NC = 2
NS = 16
LW = 128

NB = 1024


def _sc_mesh():
    return plsc.VectorSubcoreMesh(core_axis_name="c", subcore_axis_name="s")


def _make_deg_kernel(tot_w, n_pad):
    wpw = tot_w // (NC * NS)
    chunks = 23
    cw = wpw // chunks
    assert cw * chunks == wpw
    rpt = n_pad // NS

    @functools.partial(
        pl.kernel,
        mesh=_sc_mesh(),
        out_type=jax.ShapeDtypeStruct((NC, n_pad), jnp.float32),
        scratch_types=[
            pltpu.VMEM((cw, LW), jnp.int32),
            pltpu.VMEM((LW,), jnp.float32),
            pltpu.VMEM_SHARED((n_pad,), jnp.float32),
        ],
    )
    def deg_kernel(dst_hbm, zeros1_hbm, ones_hbm, out_hbm, dbuf, ones_v, dacc):
        c = lax.axis_index("c")
        s = lax.axis_index("s")
        wid = c * NS + s
        row0 = wid * wpw
        pltpu.sync_copy(ones_hbm, ones_v)
        pltpu.sync_copy(zeros1_hbm.at[pl.ds(s * rpt, rpt)],
                        dacc.at[pl.ds(s * rpt, rpt)])
        plsc.subcore_barrier()

        def chunk_body(ch, _):
            pltpu.sync_copy(dst_hbm.at[pl.ds(row0 + ch * cw, cw)], dbuf)

            def win_body(j, _):
                pltpu.sync_copy(ones_v, dacc.at[dbuf.at[j]], add=True)
                return 0

            lax.fori_loop(0, cw, win_body, 0)
            return 0

        lax.fori_loop(0, chunks, chunk_body, 0)
        plsc.subcore_barrier()
        pltpu.sync_copy(dacc.at[pl.ds(s * rpt, rpt)],
                        out_hbm.at[c, pl.ds(s * rpt, rpt)])

    return deg_kernel


def _make_scat_kernel(tot_w, n_pad):
    wpw = tot_w // (NC * NS)
    chunks = 23
    cw = wpw // chunks
    assert cw * chunks == wpw
    rpt = n_pad // NS

    @functools.partial(
        pl.kernel,
        mesh=_sc_mesh(),
        out_type=jax.ShapeDtypeStruct((NC, n_pad, 16), jnp.float32),
        scratch_types=[
            pltpu.VMEM((cw, LW), jnp.int32),
            pltpu.VMEM((cw, LW), jnp.int32),
            pltpu.VMEM((LW, 16), jnp.float32),
            pltpu.VMEM_SHARED((n_pad, 16), jnp.float32),
        ],
    )
    def scat_kernel(src_hbm, dst_hbm, u_hbm, zeros16_hbm, out_hbm,
                    sbuf, dbuf, gbuf, acc):
        c = lax.axis_index("c")
        s = lax.axis_index("s")
        wid = c * NS + s
        row0 = wid * wpw
        pltpu.sync_copy(zeros16_hbm.at[pl.ds(s * rpt, rpt)],
                        acc.at[pl.ds(s * rpt, rpt)])
        plsc.subcore_barrier()

        def chunk_body(ch, _):
            pltpu.sync_copy(src_hbm.at[pl.ds(row0 + ch * cw, cw)], sbuf)
            pltpu.sync_copy(dst_hbm.at[pl.ds(row0 + ch * cw, cw)], dbuf)

            def win_body(j, _):
                pltpu.sync_copy(u_hbm.at[sbuf.at[j]], gbuf)
                pltpu.sync_copy(gbuf, acc.at[dbuf.at[j]], add=True)
                return 0

            lax.fori_loop(0, cw, win_body, 0)
            return 0

        lax.fori_loop(0, chunks, chunk_body, 0)
        plsc.subcore_barrier()
        pltpu.sync_copy(acc.at[pl.ds(s * rpt, rpt)],
                        out_hbm.at[c, pl.ds(s * rpt, rpt)])

    return scat_kernel


def _k1_body(deg_ref, x_ref, w1_ref, dinv_ref, u1_ref):
    deg = deg_ref[0] + deg_ref[1] + 1.0
    dinv = lax.rsqrt(deg)
    x = x_ref[...]
    w1 = w1_ref[...]
    h = (x[:, 0:1] * w1[0:1, :] + x[:, 1:2] * w1[1:2, :]
         + x[:, 2:3] * w1[2:3, :])
    dinv_ref[...] = dinv
    u1_ref[...] = dinv * h


def _call_k1(n, n_pad, degp, x, w1):
    grid = (n + NB - 1) // NB
    return pl.pallas_call(
        _k1_body,
        grid=(grid,),
        in_specs=[
            pl.BlockSpec((NC, NB, 1), lambda i: (0, i, 0)),
            pl.BlockSpec((NB, 3), lambda i: (i, 0)),
            pl.BlockSpec((3, 16), lambda i: (0, 0)),
        ],
        out_specs=[
            pl.BlockSpec((NB, 1), lambda i: (i, 0)),
            pl.BlockSpec((NB, 16), lambda i: (i, 0)),
        ],
        out_shape=[
            jax.ShapeDtypeStruct((n, 1), jnp.float32),
            jax.ShapeDtypeStruct((n, 16), jnp.float32),
        ],
    )(degp.reshape(NC, n_pad, 1), x, w1)


def _k3_body(s1_ref, u1_ref, dinv_ref, b1_ref, w2_ref, u2a_ref, u2b_ref):
    dinv = dinv_ref[...]
    h = dinv * (s1_ref[0] + s1_ref[1] + u1_ref[...]) + b1_ref[...]
    h = jnp.maximum(h, 0.0)
    t = jnp.dot(h, w2_ref[...], preferred_element_type=jnp.float32)
    u2 = dinv * t
    u2a_ref[...] = u2[:, :16]
    u2b_ref[...] = u2[:, 16:]


def _call_k3(n, s1, u1, dinv, b1, w2):
    grid = (n + NB - 1) // NB
    return pl.pallas_call(
        _k3_body,
        grid=(grid,),
        in_specs=[
            pl.BlockSpec((NC, NB, 16), lambda i: (0, i, 0)),
            pl.BlockSpec((NB, 16), lambda i: (i, 0)),
            pl.BlockSpec((NB, 1), lambda i: (i, 0)),
            pl.BlockSpec((1, 16), lambda i: (0, 0)),
            pl.BlockSpec((16, 32), lambda i: (0, 0)),
        ],
        out_specs=[
            pl.BlockSpec((NB, 16), lambda i: (i, 0)),
            pl.BlockSpec((NB, 16), lambda i: (i, 0)),
        ],
        out_shape=[
            jax.ShapeDtypeStruct((n, 16), jnp.float32),
            jax.ShapeDtypeStruct((n, 16), jnp.float32),
        ],
    )(s1, u1, dinv, b1, w2)


def _k5_body(n, num_g, grid, s2a_ref, s2b_ref, u2a_ref, u2b_ref, dinv_ref,
             b2_ref, batch_ref, wlin_ref, blin_ref, out_ref, acc, cnt):
    i = pl.program_id(0)
    dinv = dinv_ref[...]
    ha = s2a_ref[0] + s2a_ref[1] + u2a_ref[...]
    hb = s2b_ref[0] + s2b_ref[1] + u2b_ref[...]
    h = dinv * jnp.concatenate([ha, hb], axis=1) + b2_ref[...]
    h = jnp.maximum(h, 0.0)
    row = i * NB + lax.broadcasted_iota(jnp.int32, (NB, 1), 0)
    valid = row < n
    cols = lax.broadcasted_iota(jnp.int32, (NB, num_g), 1)
    onehot = jnp.where((batch_ref[...] == cols) & valid, 1.0, 0.0)

    @pl.when(i == 0)
    def _():
        acc[...] = jnp.zeros_like(acc)
        cnt[...] = jnp.zeros_like(cnt)

    acc[...] += lax.dot_general(onehot, h, (((0,), (0,)), ((), ())),
                                preferred_element_type=jnp.float32)
    cnt[...] += lax.dot_general(onehot, jnp.ones((NB, 1), jnp.float32),
                                (((0,), (0,)), ((), ())),
                                preferred_element_type=jnp.float32)

    @pl.when(i == grid - 1)
    def _():
        g = acc[...] / jnp.maximum(cnt[...], 1.0)
        out_ref[...] = jnp.dot(g, wlin_ref[...],
                               preferred_element_type=jnp.float32) + blin_ref[...]


def _call_k5(n, num_g, s2a, s2b, u2a, u2b, dinv, b2, batch, wlin, blin):
    grid = (n + NB - 1) // NB
    body = functools.partial(_k5_body, n, num_g, grid)
    return pl.pallas_call(
        body,
        grid=(grid,),
        in_specs=[
            pl.BlockSpec((NC, NB, 16), lambda i: (0, i, 0)),
            pl.BlockSpec((NC, NB, 16), lambda i: (0, i, 0)),
            pl.BlockSpec((NB, 16), lambda i: (i, 0)),
            pl.BlockSpec((NB, 16), lambda i: (i, 0)),
            pl.BlockSpec((NB, 1), lambda i: (i, 0)),
            pl.BlockSpec((1, 32), lambda i: (0, 0)),
            pl.BlockSpec((NB, 1), lambda i: (i, 0)),
            pl.BlockSpec((32, 3), lambda i: (0, 0)),
            pl.BlockSpec((1, 3), lambda i: (0, 0)),
        ],
        out_specs=pl.BlockSpec((num_g, 3), lambda i: (0, 0)),
        out_shape=jax.ShapeDtypeStruct((num_g, 3), jnp.float32),
        scratch_shapes=[
            pltpu.VMEM((num_g, 32), jnp.float32),
            pltpu.VMEM((num_g, 1), jnp.float32),
        ],
    )(s2a, s2b, u2a, u2b, dinv, b2, batch, wlin, blin)


def kernel(x, edge_index, batch, W1, b1, W2, b2, Wlin, blin):
    n = x.shape[0]
    e = edge_index.shape[1]
    num_g = 512

    group = LW * NC * NS * 23
    tot_w = ((e + group - 1) // group) * group // LW
    e_pad = tot_w * LW
    n_pad = ((n + 16 * NS) // (16 * NS)) * (16 * NS)

    src = edge_index[0]
    dst = edge_index[1]
    pad = e_pad - e
    srcp = jnp.concatenate(
        [src, jnp.zeros((pad,), jnp.int32)]).reshape(tot_w, LW)
    dstp = jnp.concatenate(
        [dst, jnp.full((pad,), n, jnp.int32)]).reshape(tot_w, LW)
    zeros1 = jnp.zeros((n_pad,), jnp.float32)
    zeros16 = jnp.zeros((n_pad, 16), jnp.float32)
    ones128 = jnp.ones((LW,), jnp.float32)

    deg_k = _make_deg_kernel(tot_w, n_pad)
    scat_k = _make_scat_kernel(tot_w, n_pad)

    degp = deg_k(dstp, zeros1, ones128)
    dinv, u1 = _call_k1(n, n_pad, degp, x, W1)
    s1 = scat_k(srcp, dstp, u1, zeros16)
    u2a, u2b = _call_k3(n, s1, u1, dinv, b1.reshape(1, 16), W2)
    s2a = scat_k(srcp, dstp, u2a, zeros16)
    s2b = scat_k(srcp, dstp, u2b, zeros16)
    return _call_k5(n, num_g, s2a, s2b, u2a, u2b, dinv,
                    b2.reshape(1, 32), batch.reshape(n, 1), Wlin,
                    blin.reshape(1, 3))


# speedup vs baseline: 20.7945x; 20.7945x over previous
import functools

import jax
import jax.numpy as jnp
from jax import lax
from jax.experimental import pallas as pl
from jax.experimental.pallas import tpu as pltpu
from jax.experimental.pallas import tpu_sc as plsc

NC = 2
NS = 16
LW = 128

NB = 1024


def _sc_mesh():
    return plsc.VectorSubcoreMesh(core_axis_name="c", subcore_axis_name="s")


def _make_deg_kernel(tot_w, n_pad):
    wpw = tot_w // (NC * NS)
    cw = 32
    chunks = wpw // cw
    assert cw * chunks == wpw
    rpt = n_pad // NS
    wrpt = rpt // LW
    assert wrpt * LW == rpt

    @functools.partial(
        pl.kernel,
        mesh=_sc_mesh(),
        out_type=jax.ShapeDtypeStruct((NC, NS, wrpt, LW), jnp.float32),
        scratch_types=[
            pltpu.VMEM((cw, LW), jnp.int32),
            pltpu.VMEM((LW,), jnp.float32),
            pltpu.VMEM((LW,), jnp.float32),
            pltpu.VMEM((wrpt, LW), jnp.int32),
            pltpu.VMEM((wrpt, LW), jnp.float32),
            pltpu.VMEM_SHARED((n_pad,), jnp.float32),
        ],
    )
    def deg_kernel(dst_hbm, iota_hbm, ones_hbm, zeros_hbm, out_hbm,
                   dbuf, ones_v, zeros_v, iota_v, dump_v, dacc):
        c = lax.axis_index("c")
        s = lax.axis_index("s")
        wid = c * NS + s
        row0 = wid * wpw
        pltpu.sync_copy(ones_hbm, ones_v)
        pltpu.sync_copy(zeros_hbm, zeros_v)
        pltpu.sync_copy(iota_hbm.at[s], iota_v)

        def zero_body(j, _):
            pltpu.sync_copy(zeros_v, dacc.at[iota_v.at[j]])
            return 0

        lax.fori_loop(0, wrpt, zero_body, 0)
        plsc.subcore_barrier()

        def chunk_body(ch, _):
            pltpu.sync_copy(dst_hbm.at[pl.ds(row0 + ch * cw, cw)], dbuf)

            def win_body(j, _):
                pltpu.sync_copy(ones_v, dacc.at[dbuf.at[j]], add=True)
                return 0

            lax.fori_loop(0, cw, win_body, 0)
            return 0

        lax.fori_loop(0, chunks, chunk_body, 0)
        plsc.subcore_barrier()

        def dump_body(j, _):
            pltpu.sync_copy(dacc.at[iota_v.at[j]], dump_v.at[j])
            return 0

        lax.fori_loop(0, wrpt, dump_body, 0)
        pltpu.sync_copy(dump_v, out_hbm.at[c, s])

    return deg_kernel


def _make_scat_kernel(tot_w, n_pad):
    wpw = tot_w // (NC * NS)
    cw = 32
    chunks = wpw // cw
    assert cw * chunks == wpw
    rpt = n_pad // NS

    @functools.partial(
        pl.kernel,
        mesh=_sc_mesh(),
        compiler_params=pltpu.CompilerParams(use_tc_tiling_on_sc=False),
        out_type=jax.ShapeDtypeStruct((NC, n_pad, 16), jnp.float32),
        scratch_types=[
            pltpu.VMEM((cw, LW), jnp.int32),
            pltpu.VMEM((cw, LW), jnp.int32),
            pltpu.VMEM((LW, 16), jnp.float32),
            pltpu.VMEM_SHARED((n_pad, 16), jnp.float32),
        ],
    )
    def scat_kernel(src_hbm, dst_hbm, u_hbm, zeros16_hbm, out_hbm,
                    sbuf, dbuf, gbuf, acc):
        c = lax.axis_index("c")
        s = lax.axis_index("s")
        wid = c * NS + s
        row0 = wid * wpw
        pltpu.sync_copy(zeros16_hbm.at[pl.ds(s * rpt, rpt)],
                        acc.at[pl.ds(s * rpt, rpt)])
        plsc.subcore_barrier()

        def chunk_body(ch, _):
            pltpu.sync_copy(src_hbm.at[pl.ds(row0 + ch * cw, cw)], sbuf)
            pltpu.sync_copy(dst_hbm.at[pl.ds(row0 + ch * cw, cw)], dbuf)

            def win_body(j, _):
                pltpu.sync_copy(u_hbm.at[sbuf.at[j]], gbuf)
                pltpu.sync_copy(gbuf, acc.at[dbuf.at[j]], add=True)
                return 0

            lax.fori_loop(0, cw, win_body, 0)
            return 0

        lax.fori_loop(0, chunks, chunk_body, 0)
        plsc.subcore_barrier()
        pltpu.sync_copy(acc.at[pl.ds(s * rpt, rpt)],
                        out_hbm.at[c, pl.ds(s * rpt, rpt)])

    return scat_kernel


def _k1_body(deg_ref, x_ref, w1_ref, dinv_ref, u1_ref):
    deg = deg_ref[0] + deg_ref[1] + 1.0
    dinv = lax.rsqrt(deg)
    x = x_ref[...]
    w1 = w1_ref[...]
    h = (x[:, 0:1] * w1[0:1, :] + x[:, 1:2] * w1[1:2, :]
         + x[:, 2:3] * w1[2:3, :])
    dinv_ref[...] = dinv
    u1_ref[...] = dinv * h


def _call_k1(n, n_pad, degp, x, w1):
    grid = (n + NB - 1) // NB
    return pl.pallas_call(
        _k1_body,
        grid=(grid,),
        in_specs=[
            pl.BlockSpec((NC, NB, 1), lambda i: (0, i, 0)),
            pl.BlockSpec((NB, 3), lambda i: (i, 0)),
            pl.BlockSpec((3, 16), lambda i: (0, 0)),
        ],
        out_specs=[
            pl.BlockSpec((NB, 1), lambda i: (i, 0)),
            pl.BlockSpec((NB, 16), lambda i: (i, 0)),
        ],
        out_shape=[
            jax.ShapeDtypeStruct((n, 1), jnp.float32),
            jax.ShapeDtypeStruct((n, 16), jnp.float32),
        ],
    )(degp.reshape(NC, n_pad, 1), x, w1)


def _k3_body(s1_ref, u1_ref, dinv_ref, b1_ref, w2_ref, u2a_ref, u2b_ref):
    dinv = dinv_ref[...]
    h = dinv * (s1_ref[0] + s1_ref[1] + u1_ref[...]) + b1_ref[...]
    h = jnp.maximum(h, 0.0)
    t = jnp.dot(h, w2_ref[...], preferred_element_type=jnp.float32)
    u2 = dinv * t
    u2a_ref[...] = u2[:, :16]
    u2b_ref[...] = u2[:, 16:]


def _call_k3(n, s1, u1, dinv, b1, w2):
    grid = (n + NB - 1) // NB
    return pl.pallas_call(
        _k3_body,
        grid=(grid,),
        in_specs=[
            pl.BlockSpec((NC, NB, 16), lambda i: (0, i, 0)),
            pl.BlockSpec((NB, 16), lambda i: (i, 0)),
            pl.BlockSpec((NB, 1), lambda i: (i, 0)),
            pl.BlockSpec((1, 16), lambda i: (0, 0)),
            pl.BlockSpec((16, 32), lambda i: (0, 0)),
        ],
        out_specs=[
            pl.BlockSpec((NB, 16), lambda i: (i, 0)),
            pl.BlockSpec((NB, 16), lambda i: (i, 0)),
        ],
        out_shape=[
            jax.ShapeDtypeStruct((n, 16), jnp.float32),
            jax.ShapeDtypeStruct((n, 16), jnp.float32),
        ],
    )(s1, u1, dinv, b1, w2)


def _k5_body(n, num_g, grid, s2a_ref, s2b_ref, u2a_ref, u2b_ref, dinv_ref,
             b2_ref, batch_ref, wlin_ref, blin_ref, out_ref, acc, cnt):
    i = pl.program_id(0)
    dinv = dinv_ref[...]
    ha = s2a_ref[0] + s2a_ref[1] + u2a_ref[...]
    hb = s2b_ref[0] + s2b_ref[1] + u2b_ref[...]
    h = dinv * jnp.concatenate([ha, hb], axis=1) + b2_ref[...]
    h = jnp.maximum(h, 0.0)
    row = i * NB + lax.broadcasted_iota(jnp.int32, (NB, 1), 0)
    valid = row < n
    cols = lax.broadcasted_iota(jnp.int32, (NB, num_g), 1)
    onehot = jnp.where((batch_ref[...] == cols) & valid, 1.0, 0.0)

    @pl.when(i == 0)
    def _():
        acc[...] = jnp.zeros_like(acc)
        cnt[...] = jnp.zeros_like(cnt)

    acc[...] += lax.dot_general(onehot, h, (((0,), (0,)), ((), ())),
                                preferred_element_type=jnp.float32)
    cnt[...] += lax.dot_general(onehot, jnp.ones((NB, 1), jnp.float32),
                                (((0,), (0,)), ((), ())),
                                preferred_element_type=jnp.float32)

    @pl.when(i == grid - 1)
    def _():
        g = acc[...] / jnp.maximum(cnt[...], 1.0)
        out_ref[...] = jnp.dot(g, wlin_ref[...],
                               preferred_element_type=jnp.float32) + blin_ref[...]


def _call_k5(n, num_g, s2a, s2b, u2a, u2b, dinv, b2, batch, wlin, blin):
    grid = (n + NB - 1) // NB
    body = functools.partial(_k5_body, n, num_g, grid)
    return pl.pallas_call(
        body,
        grid=(grid,),
        in_specs=[
            pl.BlockSpec((NC, NB, 16), lambda i: (0, i, 0)),
            pl.BlockSpec((NC, NB, 16), lambda i: (0, i, 0)),
            pl.BlockSpec((NB, 16), lambda i: (i, 0)),
            pl.BlockSpec((NB, 16), lambda i: (i, 0)),
            pl.BlockSpec((NB, 1), lambda i: (i, 0)),
            pl.BlockSpec((1, 32), lambda i: (0, 0)),
            pl.BlockSpec((NB, 1), lambda i: (i, 0)),
            pl.BlockSpec((32, 3), lambda i: (0, 0)),
            pl.BlockSpec((1, 3), lambda i: (0, 0)),
        ],
        out_specs=pl.BlockSpec((num_g, 3), lambda i: (0, 0)),
        out_shape=jax.ShapeDtypeStruct((num_g, 3), jnp.float32),
        scratch_shapes=[
            pltpu.VMEM((num_g, 32), jnp.float32),
            pltpu.VMEM((num_g, 1), jnp.float32),
        ],
    )(s2a, s2b, u2a, u2b, dinv, b2, batch, wlin, blin)


def kernel(x, edge_index, batch, W1, b1, W2, b2, Wlin, blin):
    n = x.shape[0]
    e = edge_index.shape[1]
    num_g = 512

    group = LW * NC * NS * 32
    tot_w = ((e + group - 1) // group) * group // LW
    e_pad = tot_w * LW
    n_pad = ((n + NS * LW) // (NS * LW)) * (NS * LW)

    src = edge_index[0]
    dst = edge_index[1]
    pad = e_pad - e
    srcp = jnp.concatenate(
        [src, jnp.zeros((pad,), jnp.int32)]).reshape(tot_w, LW)
    dstp = jnp.concatenate(
        [dst, jnp.full((pad,), n, jnp.int32)]).reshape(tot_w, LW)
    zeros16 = jnp.zeros((n_pad, 16), jnp.float32)
    ones128 = jnp.ones((LW,), jnp.float32)
    zeros128 = jnp.zeros((LW,), jnp.float32)
    iota_nm = jnp.arange(n_pad, dtype=jnp.int32).reshape(NS, n_pad // NS // LW, LW)

    deg_k = _make_deg_kernel(tot_w, n_pad)
    scat_k = _make_scat_kernel(tot_w, n_pad)

    degp = deg_k(dstp, iota_nm, ones128, zeros128).reshape(NC, n_pad)
    dinv, u1 = _call_k1(n, n_pad, degp, x, W1)
    s1 = scat_k(srcp, dstp, u1, zeros16)
    u2a, u2b = _call_k3(n, s1, u1, dinv, b1.reshape(1, 16), W2)
    s2a = scat_k(srcp, dstp, u2a, zeros16)
    s2b = scat_k(srcp, dstp, u2b, zeros16)
    return _call_k5(n, num_g, s2a, s2b, u2a, u2b, dinv,
                    b2.reshape(1, 32), batch.reshape(n, 1), Wlin,
                    blin.reshape(1, 3))


# speedup vs baseline: 46.2791x; 2.2255x over previous
import functools

import jax
import jax.numpy as jnp
from jax import lax
from jax.experimental import pallas as pl
from jax.experimental.pallas import tpu as pltpu
from jax.experimental.pallas import tpu_sc as plsc

NC = 2
NS = 16
LW = 128
CW = 16
CWS = 4

NB = 2048


def _sc_mesh():
    return plsc.VectorSubcoreMesh(core_axis_name="c", subcore_axis_name="s")


def _make_deg_kernel(tot_w, n_pad):
    wpw = tot_w // (NC * NS)
    chunks = wpw // CW
    assert CW * chunks == wpw and chunks >= 2
    rpt = n_pad // NS
    wrpt = rpt // LW
    assert wrpt * LW == rpt

    @functools.partial(
        pl.kernel,
        mesh=_sc_mesh(),
        out_type=jax.ShapeDtypeStruct((NC, NS, wrpt, LW), jnp.float32),
        scratch_types=[
            pltpu.VMEM((2, CW, LW), jnp.int32),
            pltpu.VMEM((LW,), jnp.float32),
            pltpu.VMEM((LW,), jnp.float32),
            pltpu.VMEM((wrpt, LW), jnp.int32),
            pltpu.VMEM((wrpt, LW), jnp.float32),
            pltpu.VMEM_SHARED((n_pad,), jnp.float32),
            pltpu.SemaphoreType.DMA((2,)),
        ],
    )
    def deg_kernel(dst_hbm, iota_hbm, ones_hbm, zeros_hbm, out_hbm,
                   dbuf, ones_v, zeros_v, iota_v, dump_v, dacc, ssem):
        c = lax.axis_index("c")
        s = lax.axis_index("s")
        wid = c * NS + s
        row0 = wid * wpw
        pltpu.sync_copy(ones_hbm, ones_v)
        pltpu.sync_copy(zeros_hbm, zeros_v)
        pltpu.sync_copy(iota_hbm.at[s], iota_v)

        def zero_body(j, _):
            pltpu.sync_copy(zeros_v, dacc.at[iota_v.at[j]])
            return 0

        lax.fori_loop(0, wrpt, zero_body, 0)
        plsc.subcore_barrier()

        def drain_slot(sl):
            def d(j, _):
                pltpu.make_async_copy(
                    ones_v, dacc.at[dbuf.at[sl, j]], ssem.at[sl]).wait()
                return 0
            lax.fori_loop(0, CW, d, 0)

        def chunk_body(ch, _):
            a = jnp.bitwise_and(ch, 1)

            @pl.when(ch >= 2)
            def _():
                drain_slot(a)

            pltpu.sync_copy(dst_hbm.at[pl.ds(row0 + ch * CW, CW)], dbuf.at[a])

            def fire(j, _):
                pltpu.async_copy(ones_v, dacc.at[dbuf.at[a, j]], ssem.at[a],
                                 add=True)
                return 0

            lax.fori_loop(0, CW, fire, 0)
            return 0

        lax.fori_loop(0, chunks, chunk_body, 0)
        drain_slot((chunks - 1) % 2)
        drain_slot((chunks - 2) % 2)
        plsc.subcore_barrier()

        def dump_body(j, _):
            pltpu.sync_copy(dacc.at[iota_v.at[j]], dump_v.at[j])
            return 0

        lax.fori_loop(0, wrpt, dump_body, 0)
        pltpu.sync_copy(dump_v, out_hbm.at[c, s])

    return deg_kernel


def _make_scat_kernel(tot_w, n_pad):
    wpw = tot_w // (NC * NS)
    chunks = wpw // CWS
    assert CWS * chunks == wpw and chunks >= 2
    rpt = n_pad // NS

    @functools.partial(
        pl.kernel,
        mesh=_sc_mesh(),
        compiler_params=pltpu.CompilerParams(use_tc_tiling_on_sc=False),
        out_type=jax.ShapeDtypeStruct((NC, n_pad, 16), jnp.float32),
        scratch_types=[
            pltpu.VMEM((2, CWS, LW), jnp.int32),
            pltpu.VMEM((2, CWS, LW), jnp.int32),
            pltpu.VMEM((2, CWS, LW, 16), jnp.float32),
            pltpu.VMEM_SHARED((n_pad, 16), jnp.float32),
            pltpu.SemaphoreType.DMA((2,)),
            pltpu.SemaphoreType.DMA((2,)),
        ],
    )
    def scat_kernel(src_hbm, dst_hbm, u_hbm, zeros16_hbm, out_hbm,
                    sbuf, dbuf, gbuf, acc, gsem, ssem):
        c = lax.axis_index("c")
        s = lax.axis_index("s")
        row0 = (c * NS + s) * wpw
        u_src = u_hbm
        pltpu.sync_copy(zeros16_hbm.at[pl.ds(s * rpt, rpt)],
                        acc.at[pl.ds(s * rpt, rpt)])
        plsc.subcore_barrier()

        def drain_gather(sl):
            def d(j, _):
                pltpu.make_async_copy(
                    u_src.at[sbuf.at[sl, j]], gbuf.at[sl, j],
                    gsem.at[sl]).wait()
                return 0
            lax.fori_loop(0, CWS, d, 0)

        def fire_scatter(sl):
            def f(j, _):
                pltpu.async_copy(gbuf.at[sl, j], acc.at[dbuf.at[sl, j]],
                                 ssem.at[sl], add=True)
                return 0
            lax.fori_loop(0, CWS, f, 0)

        def drain_scatter(sl):
            def d(j, _):
                pltpu.make_async_copy(
                    gbuf.at[sl, j], acc.at[dbuf.at[sl, j]],
                    ssem.at[sl]).wait()
                return 0
            lax.fori_loop(0, CWS, d, 0)

        def chunk_body(ch, _):
            a = jnp.bitwise_and(ch, 1)
            b = 1 - a

            @pl.when(ch >= 2)
            def _():
                drain_scatter(a)

            pltpu.sync_copy(src_hbm.at[pl.ds(row0 + ch * CWS, CWS)], sbuf.at[a])
            pltpu.sync_copy(dst_hbm.at[pl.ds(row0 + ch * CWS, CWS)], dbuf.at[a])

            def fire_gather(j, _):
                pltpu.async_copy(u_src.at[sbuf.at[a, j]], gbuf.at[a, j],
                                 gsem.at[a])
                return 0

            lax.fori_loop(0, CWS, fire_gather, 0)

            @pl.when(ch >= 1)
            def _():
                drain_gather(b)
                fire_scatter(b)

            return 0

        lax.fori_loop(0, chunks, chunk_body, 0)
        last = (chunks - 1) % 2
        drain_gather(last)
        fire_scatter(last)
        drain_scatter((chunks - 2) % 2)
        drain_scatter(last)
        plsc.subcore_barrier()
        pltpu.sync_copy(acc.at[pl.ds(s * rpt, rpt)],
                        out_hbm.at[c, pl.ds(s * rpt, rpt)])

    return scat_kernel


def _k1_body(deg_ref, x_ref, w1_ref, dinv_ref, u1_ref):
    deg = deg_ref[0] + deg_ref[1] + 1.0
    dinv = lax.rsqrt(deg)
    x = x_ref[...]
    w1 = w1_ref[...]
    h = (x[:, 0:1] * w1[0:1, :] + x[:, 1:2] * w1[1:2, :]
         + x[:, 2:3] * w1[2:3, :])
    dinv_ref[...] = dinv
    u1_ref[...] = dinv * h


def _call_k1(n, n_pad, degp, x, w1):
    grid = (n + NB - 1) // NB
    return pl.pallas_call(
        _k1_body,
        grid=(grid,),
        in_specs=[
            pl.BlockSpec((NC, NB, 1), lambda i: (0, i, 0)),
            pl.BlockSpec((NB, 3), lambda i: (i, 0)),
            pl.BlockSpec((3, 16), lambda i: (0, 0)),
        ],
        out_specs=[
            pl.BlockSpec((NB, 1), lambda i: (i, 0)),
            pl.BlockSpec((NB, 16), lambda i: (i, 0)),
        ],
        out_shape=[
            jax.ShapeDtypeStruct((n, 1), jnp.float32),
            jax.ShapeDtypeStruct((n, 16), jnp.float32),
        ],
    )(degp.reshape(NC, n_pad, 1), x, w1)


def _k3_body(s1_ref, u1_ref, dinv_ref, b1_ref, w2_ref, u2a_ref, u2b_ref):
    dinv = dinv_ref[...]
    h = dinv * (s1_ref[0] + s1_ref[1] + u1_ref[...]) + b1_ref[...]
    h = jnp.maximum(h, 0.0)
    t = jnp.dot(h, w2_ref[...], preferred_element_type=jnp.float32)
    u2 = dinv * t
    u2a_ref[...] = u2[:, :16]
    u2b_ref[...] = u2[:, 16:]


def _call_k3(n, s1, u1, dinv, b1, w2):
    grid = (n + NB - 1) // NB
    return pl.pallas_call(
        _k3_body,
        grid=(grid,),
        in_specs=[
            pl.BlockSpec((NC, NB, 16), lambda i: (0, i, 0)),
            pl.BlockSpec((NB, 16), lambda i: (i, 0)),
            pl.BlockSpec((NB, 1), lambda i: (i, 0)),
            pl.BlockSpec((1, 16), lambda i: (0, 0)),
            pl.BlockSpec((16, 32), lambda i: (0, 0)),
        ],
        out_specs=[
            pl.BlockSpec((NB, 16), lambda i: (i, 0)),
            pl.BlockSpec((NB, 16), lambda i: (i, 0)),
        ],
        out_shape=[
            jax.ShapeDtypeStruct((n, 16), jnp.float32),
            jax.ShapeDtypeStruct((n, 16), jnp.float32),
        ],
    )(s1, u1, dinv, b1, w2)


def _k5_body(n, num_g, grid, s2a_ref, s2b_ref, u2a_ref, u2b_ref, dinv_ref,
             b2_ref, batch_ref, wlin_ref, blin_ref, out_ref, acc, cnt):
    i = pl.program_id(0)
    dinv = dinv_ref[...]
    ha = s2a_ref[0] + s2a_ref[1] + u2a_ref[...]
    hb = s2b_ref[0] + s2b_ref[1] + u2b_ref[...]
    h = dinv * jnp.concatenate([ha, hb], axis=1) + b2_ref[...]
    h = jnp.maximum(h, 0.0)
    row = i * NB + lax.broadcasted_iota(jnp.int32, (NB, 1), 0)
    valid = row < n
    cols = lax.broadcasted_iota(jnp.int32, (NB, num_g), 1)
    onehot = jnp.where((batch_ref[...] == cols) & valid, 1.0, 0.0)

    @pl.when(i == 0)
    def _():
        acc[...] = jnp.zeros_like(acc)
        cnt[...] = jnp.zeros_like(cnt)

    acc[...] += lax.dot_general(onehot, h, (((0,), (0,)), ((), ())),
                                preferred_element_type=jnp.float32)
    cnt[...] += lax.dot_general(onehot, jnp.ones((NB, 1), jnp.float32),
                                (((0,), (0,)), ((), ())),
                                preferred_element_type=jnp.float32)

    @pl.when(i == grid - 1)
    def _():
        g = acc[...] / jnp.maximum(cnt[...], 1.0)
        out_ref[...] = jnp.dot(g, wlin_ref[...],
                               preferred_element_type=jnp.float32) + blin_ref[...]


def _call_k5(n, num_g, s2a, s2b, u2a, u2b, dinv, b2, batch, wlin, blin):
    grid = (n + NB - 1) // NB
    body = functools.partial(_k5_body, n, num_g, grid)
    return pl.pallas_call(
        body,
        grid=(grid,),
        in_specs=[
            pl.BlockSpec((NC, NB, 16), lambda i: (0, i, 0)),
            pl.BlockSpec((NC, NB, 16), lambda i: (0, i, 0)),
            pl.BlockSpec((NB, 16), lambda i: (i, 0)),
            pl.BlockSpec((NB, 16), lambda i: (i, 0)),
            pl.BlockSpec((NB, 1), lambda i: (i, 0)),
            pl.BlockSpec((1, 32), lambda i: (0, 0)),
            pl.BlockSpec((NB, 1), lambda i: (i, 0)),
            pl.BlockSpec((32, 3), lambda i: (0, 0)),
            pl.BlockSpec((1, 3), lambda i: (0, 0)),
        ],
        out_specs=pl.BlockSpec((num_g, 3), lambda i: (0, 0)),
        out_shape=jax.ShapeDtypeStruct((num_g, 3), jnp.float32),
        scratch_shapes=[
            pltpu.VMEM((num_g, 32), jnp.float32),
            pltpu.VMEM((num_g, 1), jnp.float32),
        ],
    )(s2a, s2b, u2a, u2b, dinv, b2, batch, wlin, blin)


def kernel(x, edge_index, batch, W1, b1, W2, b2, Wlin, blin):
    n = x.shape[0]
    e = edge_index.shape[1]
    num_g = 512

    unit = LW * NC * NS * CW
    e_pad = ((e + unit - 1) // unit) * unit
    tot_w = e_pad // LW
    n_pad = ((n + NS * LW) // (NS * LW)) * (NS * LW)

    src = edge_index[0]
    dst = edge_index[1]
    pad = e_pad - e
    srcp = jnp.concatenate(
        [src, jnp.zeros((pad,), jnp.int32)]).reshape(tot_w, LW)
    dstp = jnp.concatenate(
        [dst, jnp.full((pad,), n, jnp.int32)]).reshape(tot_w, LW)
    zeros16 = jnp.zeros((n_pad, 16), jnp.float32)
    ones128 = jnp.ones((LW,), jnp.float32)
    zeros128 = jnp.zeros((LW,), jnp.float32)
    iota_nm = jnp.arange(n_pad, dtype=jnp.int32).reshape(
        NS, n_pad // NS // LW, LW)

    deg_k = _make_deg_kernel(tot_w, n_pad)
    scat_k = _make_scat_kernel(tot_w, n_pad)

    degp = deg_k(dstp, iota_nm, ones128, zeros128)
    dinv, u1 = _call_k1(n, n_pad, degp, x, W1)
    s1 = scat_k(srcp, dstp, u1, zeros16)
    u2a, u2b = _call_k3(n, s1, u1, dinv, b1.reshape(1, 16), W2)
    s2a = scat_k(srcp, dstp, u2a, zeros16)
    s2b = scat_k(srcp, dstp, u2b, zeros16)
    return _call_k5(n, num_g, s2a, s2b, u2a, u2b, dinv,
                    b2.reshape(1, 32), batch.reshape(n, 1), Wlin,
                    blin.reshape(1, 3))


# speedup vs baseline: 46.9672x; 1.0149x over previous
import functools

import jax
import jax.numpy as jnp
from jax import lax
from jax.experimental import pallas as pl
from jax.experimental.pallas import tpu as pltpu
from jax.experimental.pallas import tpu_sc as plsc

NC = 2
NS = 16
LW = 128
CW = 16
CWS = 4

NB = 2048
GW = 136


def _sc_mesh():
    return plsc.VectorSubcoreMesh(core_axis_name="c", subcore_axis_name="s")


def _make_deg_kernel(tot_w, n_pad):
    wpw = tot_w // (NC * NS)
    chunks = wpw // CW
    assert CW * chunks == wpw and chunks >= 2
    rpt = n_pad // NS
    wrpt = rpt // LW
    assert wrpt * LW == rpt

    @functools.partial(
        pl.kernel,
        mesh=_sc_mesh(),
        out_type=jax.ShapeDtypeStruct((NC, NS, wrpt, LW), jnp.float32),
        scratch_types=[
            pltpu.VMEM((2, CW, LW), jnp.int32),
            pltpu.VMEM((LW,), jnp.float32),
            pltpu.VMEM((LW,), jnp.float32),
            pltpu.VMEM((wrpt, LW), jnp.int32),
            pltpu.VMEM((wrpt, LW), jnp.float32),
            pltpu.VMEM_SHARED((n_pad,), jnp.float32),
            pltpu.SemaphoreType.DMA((2,)),
        ],
    )
    def deg_kernel(dst_hbm, iota_hbm, ones_hbm, zeros_hbm, out_hbm,
                   dbuf, ones_v, zeros_v, iota_v, dump_v, dacc, ssem):
        c = lax.axis_index("c")
        s = lax.axis_index("s")
        wid = c * NS + s
        row0 = wid * wpw
        pltpu.sync_copy(ones_hbm, ones_v)
        pltpu.sync_copy(zeros_hbm, zeros_v)
        pltpu.sync_copy(iota_hbm.at[s], iota_v)

        def zero_body(j, _):
            pltpu.sync_copy(zeros_v, dacc.at[iota_v.at[j]])
            return 0

        lax.fori_loop(0, wrpt, zero_body, 0)
        plsc.subcore_barrier()

        def drain_slot(sl):
            def d(j, _):
                pltpu.make_async_copy(
                    ones_v, dacc.at[dbuf.at[sl, j]], ssem.at[sl]).wait()
                return 0
            lax.fori_loop(0, CW, d, 0)

        def chunk_body(ch, _):
            a = jnp.bitwise_and(ch, 1)

            @pl.when(ch >= 2)
            def _():
                drain_slot(a)

            pltpu.sync_copy(dst_hbm.at[pl.ds(row0 + ch * CW, CW)], dbuf.at[a])

            def fire(j, _):
                pltpu.async_copy(ones_v, dacc.at[dbuf.at[a, j]], ssem.at[a],
                                 add=True)
                return 0

            lax.fori_loop(0, CW, fire, 0)
            return 0

        lax.fori_loop(0, chunks, chunk_body, 0)
        drain_slot((chunks - 1) % 2)
        drain_slot((chunks - 2) % 2)
        plsc.subcore_barrier()

        def dump_body(j, _):
            pltpu.sync_copy(dacc.at[iota_v.at[j]], dump_v.at[j])
            return 0

        lax.fori_loop(0, wrpt, dump_body, 0)
        pltpu.sync_copy(dump_v, out_hbm.at[c, s])

    return deg_kernel


def _make_scat_kernel(tot_w, n_pad):
    wpw = tot_w // (NC * NS)
    chunks = wpw // CWS
    assert CWS * chunks == wpw and chunks >= 2
    rpt = n_pad // NS

    @functools.partial(
        pl.kernel,
        mesh=_sc_mesh(),
        compiler_params=pltpu.CompilerParams(use_tc_tiling_on_sc=False),
        out_type=jax.ShapeDtypeStruct((NC, n_pad, 16), jnp.float32),
        scratch_types=[
            pltpu.VMEM((2, CWS, LW), jnp.int32),
            pltpu.VMEM((2, CWS, LW), jnp.int32),
            pltpu.VMEM((2, CWS, LW, 16), jnp.float32),
            pltpu.VMEM_SHARED((n_pad, 16), jnp.float32),
            pltpu.SemaphoreType.DMA((2,)),
            pltpu.SemaphoreType.DMA((2,)),
        ],
    )
    def scat_kernel(src_hbm, dst_hbm, u_hbm, zeros16_hbm, out_hbm,
                    sbuf, dbuf, gbuf, acc, gsem, ssem):
        c = lax.axis_index("c")
        s = lax.axis_index("s")
        row0 = (c * NS + s) * wpw
        u_src = u_hbm
        pltpu.sync_copy(zeros16_hbm.at[pl.ds(s * rpt, rpt)],
                        acc.at[pl.ds(s * rpt, rpt)])
        plsc.subcore_barrier()

        def drain_gather(sl):
            def d(j, _):
                pltpu.make_async_copy(
                    u_src.at[sbuf.at[sl, j]], gbuf.at[sl, j],
                    gsem.at[sl]).wait()
                return 0
            lax.fori_loop(0, CWS, d, 0)

        def fire_scatter(sl):
            def f(j, _):
                pltpu.async_copy(gbuf.at[sl, j], acc.at[dbuf.at[sl, j]],
                                 ssem.at[sl], add=True)
                return 0
            lax.fori_loop(0, CWS, f, 0)

        def drain_scatter(sl):
            def d(j, _):
                pltpu.make_async_copy(
                    gbuf.at[sl, j], acc.at[dbuf.at[sl, j]],
                    ssem.at[sl]).wait()
                return 0
            lax.fori_loop(0, CWS, d, 0)

        def chunk_body(ch, _):
            a = jnp.bitwise_and(ch, 1)
            b = 1 - a

            @pl.when(ch >= 2)
            def _():
                drain_scatter(a)

            pltpu.sync_copy(src_hbm.at[pl.ds(row0 + ch * CWS, CWS)], sbuf.at[a])
            pltpu.sync_copy(dst_hbm.at[pl.ds(row0 + ch * CWS, CWS)], dbuf.at[a])

            def fire_gather(j, _):
                pltpu.async_copy(u_src.at[sbuf.at[a, j]], gbuf.at[a, j],
                                 gsem.at[a])
                return 0

            lax.fori_loop(0, CWS, fire_gather, 0)

            @pl.when(ch >= 1)
            def _():
                drain_gather(b)
                fire_scatter(b)

            return 0

        lax.fori_loop(0, chunks, chunk_body, 0)
        last = (chunks - 1) % 2
        drain_gather(last)
        fire_scatter(last)
        drain_scatter((chunks - 2) % 2)
        drain_scatter(last)
        plsc.subcore_barrier()
        pltpu.sync_copy(acc.at[pl.ds(s * rpt, rpt)],
                        out_hbm.at[c, pl.ds(s * rpt, rpt)])

    return scat_kernel


def _k1_body(deg_ref, x_ref, w1_ref, dinv_ref, u1_ref):
    deg = deg_ref[0] + deg_ref[1] + 1.0
    dinv = lax.rsqrt(deg)
    x = x_ref[...]
    w1 = w1_ref[...]
    h = (x[:, 0:1] * w1[0:1, :] + x[:, 1:2] * w1[1:2, :]
         + x[:, 2:3] * w1[2:3, :])
    dinv16 = jnp.broadcast_to(dinv, (NB, 16))
    dinv_ref[...] = dinv16
    u1_ref[...] = dinv16 * h


def _call_k1(n, n_pad, degp, x, w1):
    grid = (n + NB - 1) // NB
    return pl.pallas_call(
        _k1_body,
        grid=(grid,),
        in_specs=[
            pl.BlockSpec((NC, NB, 1), lambda i: (0, i, 0)),
            pl.BlockSpec((NB, 3), lambda i: (i, 0)),
            pl.BlockSpec((3, 16), lambda i: (0, 0)),
        ],
        out_specs=[
            pl.BlockSpec((NB, 16), lambda i: (i, 0)),
            pl.BlockSpec((NB, 16), lambda i: (i, 0)),
        ],
        out_shape=[
            jax.ShapeDtypeStruct((n, 16), jnp.float32),
            jax.ShapeDtypeStruct((n, 16), jnp.float32),
        ],
    )(degp.reshape(NC, n_pad, 1), x, w1)


def _k3_body(s1_ref, u1_ref, dinv_ref, b1_ref, w2_ref, u2a_ref, u2b_ref):
    dinv = dinv_ref[...]
    h = dinv * (s1_ref[0] + s1_ref[1] + u1_ref[...]) + b1_ref[...]
    h = jnp.maximum(h, 0.0)
    t = jnp.dot(h, w2_ref[...], preferred_element_type=jnp.float32)
    u2 = jnp.concatenate([dinv, dinv], axis=1) * t
    u2a_ref[...] = u2[:, :16]
    u2b_ref[...] = u2[:, 16:]


def _call_k3(n, s1, u1, dinv, b1, w2):
    grid = (n + NB - 1) // NB
    return pl.pallas_call(
        _k3_body,
        grid=(grid,),
        in_specs=[
            pl.BlockSpec((NC, NB, 16), lambda i: (0, i, 0)),
            pl.BlockSpec((NB, 16), lambda i: (i, 0)),
            pl.BlockSpec((NB, 16), lambda i: (i, 0)),
            pl.BlockSpec((1, 16), lambda i: (0, 0)),
            pl.BlockSpec((16, 32), lambda i: (0, 0)),
        ],
        out_specs=[
            pl.BlockSpec((NB, 16), lambda i: (i, 0)),
            pl.BlockSpec((NB, 16), lambda i: (i, 0)),
        ],
        out_shape=[
            jax.ShapeDtypeStruct((n, 16), jnp.float32),
            jax.ShapeDtypeStruct((n, 16), jnp.float32),
        ],
    )(s1, u1, dinv, b1, w2)


def _k5_body(n, num_g, grid, s2a_ref, s2b_ref, u2a_ref, u2b_ref, dinv_ref,
             b2_ref, batch_ref, wlin_ref, blin_ref, out_ref, acc, cnt):
    i = pl.program_id(0)
    dinv = dinv_ref[...]
    ha = s2a_ref[0] + s2a_ref[1] + u2a_ref[...]
    hb = s2b_ref[0] + s2b_ref[1] + u2b_ref[...]
    h = (jnp.concatenate([dinv * ha, dinv * hb], axis=1)
         + b2_ref[...])
    h = jnp.maximum(h, 0.0)
    row = i * NB + lax.broadcasted_iota(jnp.int32, (NB, 1), 0)
    valid = row < n
    g0 = (batch_ref[0, 0] // 8) * 8
    cols = g0 + lax.broadcasted_iota(jnp.int32, (NB, GW), 1)
    onehot = jnp.where((batch_ref[...] == cols) & valid, 1.0, 0.0)

    @pl.when(i == 0)
    def _():
        acc[...] = jnp.zeros_like(acc)
        cnt[...] = jnp.zeros_like(cnt)

    acc[pl.ds(g0, GW), :] += lax.dot_general(
        onehot, h, (((0,), (0,)), ((), ())),
        preferred_element_type=jnp.float32)
    cnt[pl.ds(g0, GW), :] += lax.dot_general(
        onehot, jnp.ones((NB, 1), jnp.float32), (((0,), (0,)), ((), ())),
        preferred_element_type=jnp.float32)

    @pl.when(i == grid - 1)
    def _():
        g = acc[pl.ds(0, num_g), :] / jnp.maximum(cnt[pl.ds(0, num_g), :], 1.0)
        out_ref[...] = jnp.dot(g, wlin_ref[...],
                               preferred_element_type=jnp.float32) + blin_ref[...]


def _call_k5(n, num_g, s2a, s2b, u2a, u2b, dinv, b2, batch, wlin, blin):
    grid = (n + NB - 1) // NB
    body = functools.partial(_k5_body, n, num_g, grid)
    return pl.pallas_call(
        body,
        grid=(grid,),
        in_specs=[
            pl.BlockSpec((NC, NB, 16), lambda i: (0, i, 0)),
            pl.BlockSpec((NC, NB, 16), lambda i: (0, i, 0)),
            pl.BlockSpec((NB, 16), lambda i: (i, 0)),
            pl.BlockSpec((NB, 16), lambda i: (i, 0)),
            pl.BlockSpec((NB, 16), lambda i: (i, 0)),
            pl.BlockSpec((1, 32), lambda i: (0, 0)),
            pl.BlockSpec((NB, 1), lambda i: (i, 0)),
            pl.BlockSpec((32, 3), lambda i: (0, 0)),
            pl.BlockSpec((1, 3), lambda i: (0, 0)),
        ],
        out_specs=pl.BlockSpec((num_g, 3), lambda i: (0, 0)),
        out_shape=jax.ShapeDtypeStruct((num_g, 3), jnp.float32),
        scratch_shapes=[
            pltpu.VMEM((num_g + GW, 32), jnp.float32),
            pltpu.VMEM((num_g + GW, 1), jnp.float32),
        ],
    )(s2a, s2b, u2a, u2b, dinv, b2, batch, wlin, blin)


def kernel(x, edge_index, batch, W1, b1, W2, b2, Wlin, blin):
    n = x.shape[0]
    e = edge_index.shape[1]
    num_g = 512

    unit = LW * NC * NS * CW
    e_pad = ((e + unit - 1) // unit) * unit
    tot_w = e_pad // LW
    n_pad = ((n + NS * LW) // (NS * LW)) * (NS * LW)

    src = edge_index[0]
    dst = edge_index[1]
    pad = e_pad - e
    srcp = jnp.concatenate(
        [src, jnp.zeros((pad,), jnp.int32)]).reshape(tot_w, LW)
    dstp = jnp.concatenate(
        [dst, jnp.full((pad,), n, jnp.int32)]).reshape(tot_w, LW)
    zeros16 = jnp.zeros((n_pad, 16), jnp.float32)
    ones128 = jnp.ones((LW,), jnp.float32)
    zeros128 = jnp.zeros((LW,), jnp.float32)
    iota_nm = jnp.arange(n_pad, dtype=jnp.int32).reshape(
        NS, n_pad // NS // LW, LW)

    deg_k = _make_deg_kernel(tot_w, n_pad)
    scat_k = _make_scat_kernel(tot_w, n_pad)

    degp = deg_k(dstp, iota_nm, ones128, zeros128)
    dinv, u1 = _call_k1(n, n_pad, degp, x, W1)
    s1 = scat_k(srcp, dstp, u1, zeros16)
    u2a, u2b = _call_k3(n, s1, u1, dinv, b1.reshape(1, 16), W2)
    s2a = scat_k(srcp, dstp, u2a, zeros16)
    s2b = scat_k(srcp, dstp, u2b, zeros16)
    return _call_k5(n, num_g, s2a, s2b, u2a, u2b, dinv,
                    b2.reshape(1, 32), batch.reshape(n, 1), Wlin,
                    blin.reshape(1, 3))


# speedup vs baseline: 47.4091x; 1.0094x over previous
import functools

import jax
import jax.numpy as jnp
from jax import lax
from jax.experimental import pallas as pl
from jax.experimental.pallas import tpu as pltpu
from jax.experimental.pallas import tpu_sc as plsc

NC = 2
NS = 16
LW = 128
CW = 16
CWS = 4

NB = 4096
GW = 136


def _sc_mesh():
    return plsc.VectorSubcoreMesh(core_axis_name="c", subcore_axis_name="s")


def _make_deg_kernel(tot_w, n_pad):
    wpw = tot_w // (NC * NS)
    chunks = wpw // CW
    assert CW * chunks == wpw and chunks >= 2
    rpt = n_pad // NS
    wrpt = rpt // LW
    assert wrpt * LW == rpt

    @functools.partial(
        pl.kernel,
        mesh=_sc_mesh(),
        out_type=jax.ShapeDtypeStruct((NC, NS, wrpt, LW), jnp.float32),
        scratch_types=[
            pltpu.VMEM((2, CW, LW), jnp.int32),
            pltpu.VMEM((LW,), jnp.float32),
            pltpu.VMEM((LW,), jnp.float32),
            pltpu.VMEM((wrpt, LW), jnp.int32),
            pltpu.VMEM((wrpt, LW), jnp.float32),
            pltpu.VMEM_SHARED((n_pad,), jnp.float32),
            pltpu.SemaphoreType.DMA((2,)),
        ],
    )
    def deg_kernel(dst_hbm, iota_hbm, ones_hbm, zeros_hbm, out_hbm,
                   dbuf, ones_v, zeros_v, iota_v, dump_v, dacc, ssem):
        c = lax.axis_index("c")
        s = lax.axis_index("s")
        wid = c * NS + s
        row0 = wid * wpw
        pltpu.sync_copy(ones_hbm, ones_v)
        pltpu.sync_copy(zeros_hbm, zeros_v)
        pltpu.sync_copy(iota_hbm.at[s], iota_v)

        def zero_body(j, _):
            pltpu.sync_copy(zeros_v, dacc.at[iota_v.at[j]])
            return 0

        lax.fori_loop(0, wrpt, zero_body, 0)
        plsc.subcore_barrier()

        def drain_slot(sl):
            def d(j, _):
                pltpu.make_async_copy(
                    ones_v, dacc.at[dbuf.at[sl, j]], ssem.at[sl]).wait()
                return 0
            lax.fori_loop(0, CW, d, 0)

        def chunk_body(ch, _):
            a = jnp.bitwise_and(ch, 1)

            @pl.when(ch >= 2)
            def _():
                drain_slot(a)

            pltpu.sync_copy(dst_hbm.at[pl.ds(row0 + ch * CW, CW)], dbuf.at[a])

            def fire(j, _):
                pltpu.async_copy(ones_v, dacc.at[dbuf.at[a, j]], ssem.at[a],
                                 add=True)
                return 0

            lax.fori_loop(0, CW, fire, 0)
            return 0

        lax.fori_loop(0, chunks, chunk_body, 0)
        drain_slot((chunks - 1) % 2)
        drain_slot((chunks - 2) % 2)
        plsc.subcore_barrier()

        def dump_body(j, _):
            pltpu.sync_copy(dacc.at[iota_v.at[j]], dump_v.at[j])
            return 0

        lax.fori_loop(0, wrpt, dump_body, 0)
        pltpu.sync_copy(dump_v, out_hbm.at[c, s])

    return deg_kernel


def _make_scat_kernel(tot_w, n_pad):
    wpw = tot_w // (NC * NS)
    chunks = wpw // CWS
    assert CWS * chunks == wpw and chunks >= 2
    rpt = n_pad // NS

    @functools.partial(
        pl.kernel,
        mesh=_sc_mesh(),
        compiler_params=pltpu.CompilerParams(use_tc_tiling_on_sc=False),
        out_type=jax.ShapeDtypeStruct((NC, n_pad, 16), jnp.float32),
        scratch_types=[
            pltpu.VMEM((2, CWS, LW), jnp.int32),
            pltpu.VMEM((2, CWS, LW), jnp.int32),
            pltpu.VMEM((2, CWS, LW, 16), jnp.float32),
            pltpu.VMEM_SHARED((n_pad, 16), jnp.float32),
            pltpu.SemaphoreType.DMA((2,)),
            pltpu.SemaphoreType.DMA((2,)),
        ],
    )
    def scat_kernel(src_hbm, dst_hbm, u_hbm, zeros16_hbm, out_hbm,
                    sbuf, dbuf, gbuf, acc, gsem, ssem):
        c = lax.axis_index("c")
        s = lax.axis_index("s")
        row0 = (c * NS + s) * wpw
        u_src = u_hbm
        pltpu.sync_copy(zeros16_hbm.at[pl.ds(s * rpt, rpt)],
                        acc.at[pl.ds(s * rpt, rpt)])
        plsc.subcore_barrier()

        def drain_gather(sl):
            def d(j, _):
                pltpu.make_async_copy(
                    u_src.at[sbuf.at[sl, j]], gbuf.at[sl, j],
                    gsem.at[sl]).wait()
                return 0
            lax.fori_loop(0, CWS, d, 0)

        def fire_scatter(sl):
            def f(j, _):
                pltpu.async_copy(gbuf.at[sl, j], acc.at[dbuf.at[sl, j]],
                                 ssem.at[sl], add=True)
                return 0
            lax.fori_loop(0, CWS, f, 0)

        def drain_scatter(sl):
            def d(j, _):
                pltpu.make_async_copy(
                    gbuf.at[sl, j], acc.at[dbuf.at[sl, j]],
                    ssem.at[sl]).wait()
                return 0
            lax.fori_loop(0, CWS, d, 0)

        def chunk_body(ch, _):
            a = jnp.bitwise_and(ch, 1)
            b = 1 - a

            @pl.when(ch >= 2)
            def _():
                drain_scatter(a)

            pltpu.sync_copy(src_hbm.at[pl.ds(row0 + ch * CWS, CWS)], sbuf.at[a])
            pltpu.sync_copy(dst_hbm.at[pl.ds(row0 + ch * CWS, CWS)], dbuf.at[a])

            def fire_gather(j, _):
                pltpu.async_copy(u_src.at[sbuf.at[a, j]], gbuf.at[a, j],
                                 gsem.at[a])
                return 0

            lax.fori_loop(0, CWS, fire_gather, 0)

            @pl.when(ch >= 1)
            def _():
                drain_gather(b)
                fire_scatter(b)

            return 0

        lax.fori_loop(0, chunks, chunk_body, 0)
        last = (chunks - 1) % 2
        drain_gather(last)
        fire_scatter(last)
        drain_scatter((chunks - 2) % 2)
        drain_scatter(last)
        plsc.subcore_barrier()
        pltpu.sync_copy(acc.at[pl.ds(s * rpt, rpt)],
                        out_hbm.at[c, pl.ds(s * rpt, rpt)])

    return scat_kernel


def _k1_body(deg_ref, x_ref, w1_ref, dinv_ref, u1_ref):
    deg = deg_ref[0] + deg_ref[1] + 1.0
    dinv = lax.rsqrt(deg)
    x = x_ref[...]
    w1 = w1_ref[...]
    h = (x[:, 0:1] * w1[0:1, :] + x[:, 1:2] * w1[1:2, :]
         + x[:, 2:3] * w1[2:3, :])
    dinv16 = jnp.broadcast_to(dinv, (NB, 16))
    dinv_ref[...] = dinv16
    u1_ref[...] = dinv16 * h


def _call_k1(n, n_pad, degp, x, w1):
    grid = (n + NB - 1) // NB
    return pl.pallas_call(
        _k1_body,
        grid=(grid,),
        in_specs=[
            pl.BlockSpec((NC, NB, 1), lambda i: (0, i, 0)),
            pl.BlockSpec((NB, 3), lambda i: (i, 0)),
            pl.BlockSpec((3, 16), lambda i: (0, 0)),
        ],
        out_specs=[
            pl.BlockSpec((NB, 16), lambda i: (i, 0)),
            pl.BlockSpec((NB, 16), lambda i: (i, 0)),
        ],
        out_shape=[
            jax.ShapeDtypeStruct((n, 16), jnp.float32),
            jax.ShapeDtypeStruct((n, 16), jnp.float32),
        ],
    )(degp.reshape(NC, n_pad, 1), x, w1)


def _k3_body(s1_ref, u1_ref, dinv_ref, b1_ref, w2_ref, u2a_ref, u2b_ref):
    dinv = dinv_ref[...]
    h = dinv * (s1_ref[0] + s1_ref[1] + u1_ref[...]) + b1_ref[...]
    h = jnp.maximum(h, 0.0)
    t = jnp.dot(h, w2_ref[...], preferred_element_type=jnp.float32)
    u2 = jnp.concatenate([dinv, dinv], axis=1) * t
    u2a_ref[...] = u2[:, :16]
    u2b_ref[...] = u2[:, 16:]


def _call_k3(n, s1, u1, dinv, b1, w2):
    grid = (n + NB - 1) // NB
    return pl.pallas_call(
        _k3_body,
        grid=(grid,),
        in_specs=[
            pl.BlockSpec((NC, NB, 16), lambda i: (0, i, 0)),
            pl.BlockSpec((NB, 16), lambda i: (i, 0)),
            pl.BlockSpec((NB, 16), lambda i: (i, 0)),
            pl.BlockSpec((1, 16), lambda i: (0, 0)),
            pl.BlockSpec((16, 32), lambda i: (0, 0)),
        ],
        out_specs=[
            pl.BlockSpec((NB, 16), lambda i: (i, 0)),
            pl.BlockSpec((NB, 16), lambda i: (i, 0)),
        ],
        out_shape=[
            jax.ShapeDtypeStruct((n, 16), jnp.float32),
            jax.ShapeDtypeStruct((n, 16), jnp.float32),
        ],
    )(s1, u1, dinv, b1, w2)


def _k5_body(n, num_g, grid, s2a_ref, s2b_ref, u2a_ref, u2b_ref, dinv_ref,
             b2_ref, batch_ref, wlin_ref, blin_ref, out_ref, acc, cnt):
    i = pl.program_id(0)
    dinv = dinv_ref[...]
    ha = s2a_ref[0] + s2a_ref[1] + u2a_ref[...]
    hb = s2b_ref[0] + s2b_ref[1] + u2b_ref[...]
    h = (jnp.concatenate([dinv * ha, dinv * hb], axis=1)
         + b2_ref[...])
    h = jnp.maximum(h, 0.0)
    row = i * NB + lax.broadcasted_iota(jnp.int32, (NB, 1), 0)
    valid = row < n
    g0 = (batch_ref[0, 0] // 8) * 8
    cols = g0 + lax.broadcasted_iota(jnp.int32, (NB, GW), 1)
    onehot = jnp.where((batch_ref[...] == cols) & valid, 1.0, 0.0)

    @pl.when(i == 0)
    def _():
        acc[...] = jnp.zeros_like(acc)
        cnt[...] = jnp.zeros_like(cnt)

    acc[pl.ds(g0, GW), :] += lax.dot_general(
        onehot, h, (((0,), (0,)), ((), ())),
        preferred_element_type=jnp.float32)
    cnt[pl.ds(g0, GW), :] += lax.dot_general(
        onehot, jnp.ones((NB, 1), jnp.float32), (((0,), (0,)), ((), ())),
        preferred_element_type=jnp.float32)

    @pl.when(i == grid - 1)
    def _():
        g = acc[pl.ds(0, num_g), :] / jnp.maximum(cnt[pl.ds(0, num_g), :], 1.0)
        out_ref[...] = jnp.dot(g, wlin_ref[...],
                               preferred_element_type=jnp.float32) + blin_ref[...]


def _call_k5(n, num_g, s2a, s2b, u2a, u2b, dinv, b2, batch, wlin, blin):
    grid = (n + NB - 1) // NB
    body = functools.partial(_k5_body, n, num_g, grid)
    return pl.pallas_call(
        body,
        grid=(grid,),
        in_specs=[
            pl.BlockSpec((NC, NB, 16), lambda i: (0, i, 0)),
            pl.BlockSpec((NC, NB, 16), lambda i: (0, i, 0)),
            pl.BlockSpec((NB, 16), lambda i: (i, 0)),
            pl.BlockSpec((NB, 16), lambda i: (i, 0)),
            pl.BlockSpec((NB, 16), lambda i: (i, 0)),
            pl.BlockSpec((1, 32), lambda i: (0, 0)),
            pl.BlockSpec((NB, 1), lambda i: (i, 0)),
            pl.BlockSpec((32, 3), lambda i: (0, 0)),
            pl.BlockSpec((1, 3), lambda i: (0, 0)),
        ],
        out_specs=pl.BlockSpec((num_g, 3), lambda i: (0, 0)),
        out_shape=jax.ShapeDtypeStruct((num_g, 3), jnp.float32),
        scratch_shapes=[
            pltpu.VMEM((num_g + GW, 32), jnp.float32),
            pltpu.VMEM((num_g + GW, 1), jnp.float32),
        ],
    )(s2a, s2b, u2a, u2b, dinv, b2, batch, wlin, blin)


def kernel(x, edge_index, batch, W1, b1, W2, b2, Wlin, blin):
    n = x.shape[0]
    e = edge_index.shape[1]
    num_g = 512

    unit = LW * NC * NS * CW
    e_pad = ((e + unit - 1) // unit) * unit
    tot_w = e_pad // LW
    n_pad = ((n + NS * LW) // (NS * LW)) * (NS * LW)

    src = edge_index[0]
    dst = edge_index[1]
    pad = e_pad - e
    srcp = jnp.concatenate(
        [src, jnp.zeros((pad,), jnp.int32)]).reshape(tot_w, LW)
    dstp = jnp.concatenate(
        [dst, jnp.full((pad,), n, jnp.int32)]).reshape(tot_w, LW)
    zeros16 = jnp.zeros((n_pad, 16), jnp.float32)
    ones128 = jnp.ones((LW,), jnp.float32)
    zeros128 = jnp.zeros((LW,), jnp.float32)
    iota_nm = jnp.arange(n_pad, dtype=jnp.int32).reshape(
        NS, n_pad // NS // LW, LW)

    deg_k = _make_deg_kernel(tot_w, n_pad)
    scat_k = _make_scat_kernel(tot_w, n_pad)

    degp = deg_k(dstp, iota_nm, ones128, zeros128)
    dinv, u1 = _call_k1(n, n_pad, degp, x, W1)
    s1 = scat_k(srcp, dstp, u1, zeros16)
    u2a, u2b = _call_k3(n, s1, u1, dinv, b1.reshape(1, 16), W2)
    s2a = scat_k(srcp, dstp, u2a, zeros16)
    s2b = scat_k(srcp, dstp, u2b, zeros16)
    return _call_k5(n, num_g, s2a, s2b, u2a, u2b, dinv,
                    b2.reshape(1, 32), batch.reshape(n, 1), Wlin,
                    blin.reshape(1, 3))


# speedup vs baseline: 50.8394x; 1.0724x over previous
import functools

import jax
import jax.numpy as jnp
from jax import lax
from jax.experimental import pallas as pl
from jax.experimental.pallas import tpu as pltpu
from jax.experimental.pallas import tpu_sc as plsc

NC = 2
NS = 16
LW = 128
CW = 16
CWS = 4

NB = 4096
GW = 136


def _sc_mesh():
    return plsc.VectorSubcoreMesh(core_axis_name="c", subcore_axis_name="s")


def _make_deg_kernel(tot_w, n_pad):
    wpw = tot_w // (NC * NS)
    chunks = wpw // CW
    assert CW * chunks == wpw and chunks >= 2
    rpt = n_pad // NS
    wrpt = rpt // LW
    assert wrpt * LW == rpt

    @functools.partial(
        pl.kernel,
        mesh=_sc_mesh(),
        out_type=jax.ShapeDtypeStruct((NC, NS, wrpt, LW), jnp.float32),
        scratch_types=[
            pltpu.VMEM((2, CW, LW), jnp.int32),
            pltpu.VMEM((LW,), jnp.float32),
            pltpu.VMEM((LW,), jnp.float32),
            pltpu.VMEM((wrpt, LW), jnp.int32),
            pltpu.VMEM((wrpt, LW), jnp.float32),
            pltpu.VMEM_SHARED((n_pad,), jnp.float32),
            pltpu.SemaphoreType.DMA((2,)),
        ],
    )
    def deg_kernel(dst_hbm, iota_hbm, ones_hbm, zeros_hbm, out_hbm,
                   dbuf, ones_v, zeros_v, iota_v, dump_v, dacc, ssem):
        c = lax.axis_index("c")
        s = lax.axis_index("s")
        wid = c * NS + s
        row0 = wid * wpw
        pltpu.sync_copy(ones_hbm, ones_v)
        pltpu.sync_copy(zeros_hbm, zeros_v)
        pltpu.sync_copy(iota_hbm.at[s], iota_v)

        def zero_body(j, _):
            pltpu.sync_copy(zeros_v, dacc.at[iota_v.at[j]])
            return 0

        lax.fori_loop(0, wrpt, zero_body, 0)
        plsc.subcore_barrier()

        def drain_slot(sl):
            def d(j, _):
                pltpu.make_async_copy(
                    ones_v, dacc.at[dbuf.at[sl, j]], ssem.at[sl]).wait()
                return 0
            lax.fori_loop(0, CW, d, 0)

        def chunk_body(ch, _):
            a = jnp.bitwise_and(ch, 1)

            @pl.when(ch >= 2)
            def _():
                drain_slot(a)

            pltpu.sync_copy(dst_hbm.at[pl.ds(row0 + ch * CW, CW)], dbuf.at[a])

            def fire(j, _):
                pltpu.async_copy(ones_v, dacc.at[dbuf.at[a, j]], ssem.at[a],
                                 add=True)
                return 0

            lax.fori_loop(0, CW, fire, 0)
            return 0

        lax.fori_loop(0, chunks, chunk_body, 0)
        drain_slot((chunks - 1) % 2)
        drain_slot((chunks - 2) % 2)
        plsc.subcore_barrier()

        def dump_body(j, _):
            pltpu.sync_copy(dacc.at[iota_v.at[j]], dump_v.at[j])
            return 0

        lax.fori_loop(0, wrpt, dump_body, 0)
        pltpu.sync_copy(dump_v, out_hbm.at[c, s])

    return deg_kernel


def _make_scat_kernel(tot_w, n_pad):
    wpw = tot_w // (NC * NS)
    chunks = wpw // CWS
    assert CWS * chunks == wpw and chunks >= 2
    rpt = n_pad // NS

    @functools.partial(
        pl.kernel,
        mesh=_sc_mesh(),
        compiler_params=pltpu.CompilerParams(use_tc_tiling_on_sc=False),
        out_type=jax.ShapeDtypeStruct((NC, n_pad, 16), jnp.float32),
        scratch_types=[
            pltpu.VMEM((2, CWS, LW), jnp.int32),
            pltpu.VMEM((2, CWS, LW), jnp.int32),
            pltpu.VMEM((2, CWS, LW, 16), jnp.float32),
            pltpu.VMEM_SHARED((n_pad, 16), jnp.float32),
            pltpu.SemaphoreType.DMA((2,)),
            pltpu.SemaphoreType.DMA((2,)),
        ],
    )
    def scat_kernel(src_hbm, dst_hbm, u_hbm, zeros16_hbm, out_hbm,
                    sbuf, dbuf, gbuf, acc, gsem, ssem):
        c = lax.axis_index("c")
        s = lax.axis_index("s")
        row0 = (c * NS + s) * wpw
        u_src = u_hbm
        pltpu.sync_copy(zeros16_hbm.at[pl.ds(s * rpt, rpt)],
                        acc.at[pl.ds(s * rpt, rpt)])
        plsc.subcore_barrier()

        def drain_gather(sl):
            def d(j, _):
                pltpu.make_async_copy(
                    u_src.at[sbuf.at[sl, j]], gbuf.at[sl, j],
                    gsem.at[sl]).wait()
                return 0
            lax.fori_loop(0, CWS, d, 0)

        def fire_scatter(sl):
            def f(j, _):
                pltpu.async_copy(gbuf.at[sl, j], acc.at[dbuf.at[sl, j]],
                                 ssem.at[sl], add=True)
                return 0
            lax.fori_loop(0, CWS, f, 0)

        def drain_scatter(sl):
            def d(j, _):
                pltpu.make_async_copy(
                    gbuf.at[sl, j], acc.at[dbuf.at[sl, j]],
                    ssem.at[sl]).wait()
                return 0
            lax.fori_loop(0, CWS, d, 0)

        def chunk_body(ch, _):
            a = jnp.bitwise_and(ch, 1)
            b = 1 - a

            @pl.when(ch >= 2)
            def _():
                drain_scatter(a)

            pltpu.sync_copy(src_hbm.at[pl.ds(row0 + ch * CWS, CWS)], sbuf.at[a])
            pltpu.sync_copy(dst_hbm.at[pl.ds(row0 + ch * CWS, CWS)], dbuf.at[a])

            def fire_gather(j, _):
                pltpu.async_copy(u_src.at[sbuf.at[a, j]], gbuf.at[a, j],
                                 gsem.at[a])
                return 0

            lax.fori_loop(0, CWS, fire_gather, 0)

            @pl.when(ch >= 1)
            def _():
                drain_gather(b)
                fire_scatter(b)

            return 0

        lax.fori_loop(0, chunks, chunk_body, 0)
        last = (chunks - 1) % 2
        drain_gather(last)
        fire_scatter(last)
        drain_scatter((chunks - 2) % 2)
        drain_scatter(last)
        plsc.subcore_barrier()
        pltpu.sync_copy(acc.at[pl.ds(s * rpt, rpt)],
                        out_hbm.at[c, pl.ds(s * rpt, rpt)])

    return scat_kernel


def _k1_body(deg_ref, x_ref, w1_ref, dinv_ref, u1_ref):
    deg = deg_ref[0] + deg_ref[1] + 1.0
    dv = jnp.transpose(lax.rsqrt(deg))
    pieces = [jnp.broadcast_to(dv[:, p:p + 1], (128, 16))
              for p in range(NB // 128)]
    dinv16 = jnp.concatenate(pieces, axis=0)
    x = x_ref[...]
    w1 = w1_ref[...]
    h = (x[:, 0:1] * w1[0:1, :] + x[:, 1:2] * w1[1:2, :]
         + x[:, 2:3] * w1[2:3, :])
    dinv_ref[...] = dinv16
    u1_ref[...] = dinv16 * h


def _call_k1(n, n_pad, degp, x, w1):
    grid = (n + NB - 1) // NB
    return pl.pallas_call(
        _k1_body,
        grid=(grid,),
        in_specs=[
            pl.BlockSpec((NC, NB // 128, 128), lambda i: (0, i, 0)),
            pl.BlockSpec((NB, 3), lambda i: (i, 0)),
            pl.BlockSpec((3, 16), lambda i: (0, 0)),
        ],
        out_specs=[
            pl.BlockSpec((NB, 16), lambda i: (i, 0)),
            pl.BlockSpec((NB, 16), lambda i: (i, 0)),
        ],
        out_shape=[
            jax.ShapeDtypeStruct((n, 16), jnp.float32),
            jax.ShapeDtypeStruct((n, 16), jnp.float32),
        ],
    )(degp.reshape(NC, n_pad // 128, 128), x, w1)


def _k3_body(s1_ref, u1_ref, dinv_ref, b1_ref, w2_ref, u2a_ref, u2b_ref):
    dinv = dinv_ref[...]
    h = dinv * (s1_ref[0] + s1_ref[1] + u1_ref[...]) + b1_ref[...]
    h = jnp.maximum(h, 0.0)
    t = jnp.dot(h, w2_ref[...], preferred_element_type=jnp.float32)
    u2 = jnp.concatenate([dinv, dinv], axis=1) * t
    u2a_ref[...] = u2[:, :16]
    u2b_ref[...] = u2[:, 16:]


def _call_k3(n, s1, u1, dinv, b1, w2):
    grid = (n + NB - 1) // NB
    return pl.pallas_call(
        _k3_body,
        grid=(grid,),
        in_specs=[
            pl.BlockSpec((NC, NB, 16), lambda i: (0, i, 0)),
            pl.BlockSpec((NB, 16), lambda i: (i, 0)),
            pl.BlockSpec((NB, 16), lambda i: (i, 0)),
            pl.BlockSpec((1, 16), lambda i: (0, 0)),
            pl.BlockSpec((16, 32), lambda i: (0, 0)),
        ],
        out_specs=[
            pl.BlockSpec((NB, 16), lambda i: (i, 0)),
            pl.BlockSpec((NB, 16), lambda i: (i, 0)),
        ],
        out_shape=[
            jax.ShapeDtypeStruct((n, 16), jnp.float32),
            jax.ShapeDtypeStruct((n, 16), jnp.float32),
        ],
    )(s1, u1, dinv, b1, w2)


def _k5_body(n, num_g, grid, s2a_ref, s2b_ref, u2a_ref, u2b_ref, dinv_ref,
             b2_ref, batch_ref, wlin_ref, blin_ref, out_ref, acc, cnt):
    i = pl.program_id(0)
    dinv = dinv_ref[...]
    ha = s2a_ref[0] + s2a_ref[1] + u2a_ref[...]
    hb = s2b_ref[0] + s2b_ref[1] + u2b_ref[...]
    h = (jnp.concatenate([dinv * ha, dinv * hb], axis=1)
         + b2_ref[...])
    h = jnp.maximum(h, 0.0)
    row = i * NB + lax.broadcasted_iota(jnp.int32, (NB, 1), 0)
    valid = row < n
    g0 = (batch_ref[0, 0] // 8) * 8
    cols = g0 + lax.broadcasted_iota(jnp.int32, (NB, GW), 1)
    onehot = jnp.where((batch_ref[...] == cols) & valid, 1.0, 0.0)

    @pl.when(i == 0)
    def _():
        acc[...] = jnp.zeros_like(acc)
        cnt[...] = jnp.zeros_like(cnt)

    acc[pl.ds(g0, GW), :] += lax.dot_general(
        onehot, h, (((0,), (0,)), ((), ())),
        preferred_element_type=jnp.float32)
    cnt[pl.ds(g0, GW), :] += lax.dot_general(
        onehot, jnp.ones((NB, 1), jnp.float32), (((0,), (0,)), ((), ())),
        preferred_element_type=jnp.float32)

    @pl.when(i == grid - 1)
    def _():
        g = acc[pl.ds(0, num_g), :] / jnp.maximum(cnt[pl.ds(0, num_g), :], 1.0)
        out_ref[...] = jnp.dot(g, wlin_ref[...],
                               preferred_element_type=jnp.float32) + blin_ref[...]


def _call_k5(n, num_g, s2a, s2b, u2a, u2b, dinv, b2, batch, wlin, blin):
    grid = (n + NB - 1) // NB
    body = functools.partial(_k5_body, n, num_g, grid)
    return pl.pallas_call(
        body,
        grid=(grid,),
        in_specs=[
            pl.BlockSpec((NC, NB, 16), lambda i: (0, i, 0)),
            pl.BlockSpec((NC, NB, 16), lambda i: (0, i, 0)),
            pl.BlockSpec((NB, 16), lambda i: (i, 0)),
            pl.BlockSpec((NB, 16), lambda i: (i, 0)),
            pl.BlockSpec((NB, 16), lambda i: (i, 0)),
            pl.BlockSpec((1, 32), lambda i: (0, 0)),
            pl.BlockSpec((NB, 1), lambda i: (i, 0)),
            pl.BlockSpec((32, 3), lambda i: (0, 0)),
            pl.BlockSpec((1, 3), lambda i: (0, 0)),
        ],
        out_specs=pl.BlockSpec((num_g, 3), lambda i: (0, 0)),
        out_shape=jax.ShapeDtypeStruct((num_g, 3), jnp.float32),
        scratch_shapes=[
            pltpu.VMEM((num_g + GW, 32), jnp.float32),
            pltpu.VMEM((num_g + GW, 1), jnp.float32),
        ],
    )(s2a, s2b, u2a, u2b, dinv, b2, batch, wlin, blin)


def kernel(x, edge_index, batch, W1, b1, W2, b2, Wlin, blin):
    n = x.shape[0]
    e = edge_index.shape[1]
    num_g = 512

    unit = LW * NC * NS * CW
    e_pad = ((e + unit - 1) // unit) * unit
    tot_w = e_pad // LW
    n_pad = ((n + NS * LW) // (NS * LW)) * (NS * LW)

    src = edge_index[0]
    dst = edge_index[1]
    pad = e_pad - e
    srcp = jnp.concatenate(
        [src, jnp.zeros((pad,), jnp.int32)]).reshape(tot_w, LW)
    dstp = jnp.concatenate(
        [dst, jnp.full((pad,), n, jnp.int32)]).reshape(tot_w, LW)
    zeros16 = jnp.zeros((n_pad, 16), jnp.float32)
    ones128 = jnp.ones((LW,), jnp.float32)
    zeros128 = jnp.zeros((LW,), jnp.float32)
    iota_nm = jnp.arange(n_pad, dtype=jnp.int32).reshape(
        NS, n_pad // NS // LW, LW)

    deg_k = _make_deg_kernel(tot_w, n_pad)
    scat_k = _make_scat_kernel(tot_w, n_pad)

    degp = deg_k(dstp, iota_nm, ones128, zeros128)
    dinv, u1 = _call_k1(n, n_pad, degp, x, W1)
    s1 = scat_k(srcp, dstp, u1, zeros16)
    u2a, u2b = _call_k3(n, s1, u1, dinv, b1.reshape(1, 16), W2)
    s2a = scat_k(srcp, dstp, u2a, zeros16)
    s2b = scat_k(srcp, dstp, u2b, zeros16)
    return _call_k5(n, num_g, s2a, s2b, u2a, u2b, dinv,
                    b2.reshape(1, 32), batch.reshape(n, 1), Wlin,
                    blin.reshape(1, 3))


# speedup vs baseline: 61.7106x; 1.2138x over previous
import functools

import jax
import jax.numpy as jnp
from jax import lax
from jax.experimental import pallas as pl
from jax.experimental.pallas import tpu as pltpu
from jax.experimental.pallas import tpu_sc as plsc

NC = 2
NS = 16
LW = 128
CW = 16
CWS = 4

NB = 4096
GW = 136


def _sc_mesh():
    return plsc.VectorSubcoreMesh(core_axis_name="c", subcore_axis_name="s")


def _make_deg_kernel(tot_w, n_pad):
    wpw = tot_w // (NC * NS)
    chunks = wpw // CW
    assert CW * chunks == wpw and chunks >= 2
    rpt = n_pad // NS
    wrpt = rpt // LW
    assert wrpt * LW == rpt

    @functools.partial(
        pl.kernel,
        mesh=_sc_mesh(),
        out_type=jax.ShapeDtypeStruct((NC, NS, wrpt, LW), jnp.float32),
        scratch_types=[
            pltpu.VMEM((2, CW, LW), jnp.int32),
            pltpu.VMEM((LW,), jnp.float32),
            pltpu.VMEM((LW,), jnp.float32),
            pltpu.VMEM((wrpt, LW), jnp.int32),
            pltpu.VMEM((wrpt, LW), jnp.float32),
            pltpu.VMEM_SHARED((n_pad,), jnp.float32),
            pltpu.SemaphoreType.DMA((2,)),
        ],
    )
    def deg_kernel(dst_hbm, iota_hbm, ones_hbm, zeros_hbm, out_hbm,
                   dbuf, ones_v, zeros_v, iota_v, dump_v, dacc, ssem):
        c = lax.axis_index("c")
        s = lax.axis_index("s")
        wid = c * NS + s
        row0 = wid * wpw
        pltpu.sync_copy(ones_hbm, ones_v)
        pltpu.sync_copy(zeros_hbm, zeros_v)
        pltpu.sync_copy(iota_hbm.at[s], iota_v)

        def zero_body(j, _):
            pltpu.sync_copy(zeros_v, dacc.at[iota_v.at[j]])
            return 0

        lax.fori_loop(0, wrpt, zero_body, 0)
        plsc.subcore_barrier()

        def drain_slot(sl):
            def d(j, _):
                pltpu.make_async_copy(
                    ones_v, dacc.at[dbuf.at[sl, j]], ssem.at[sl]).wait()
                return 0
            lax.fori_loop(0, CW, d, 0)

        def chunk_body(ch, _):
            a = jnp.bitwise_and(ch, 1)

            @pl.when(ch >= 2)
            def _():
                drain_slot(a)

            pltpu.sync_copy(dst_hbm.at[pl.ds(row0 + ch * CW, CW)], dbuf.at[a])

            def fire(j, _):
                pltpu.async_copy(ones_v, dacc.at[dbuf.at[a, j]], ssem.at[a],
                                 add=True)
                return 0

            lax.fori_loop(0, CW, fire, 0)
            return 0

        lax.fori_loop(0, chunks, chunk_body, 0)
        drain_slot((chunks - 1) % 2)
        drain_slot((chunks - 2) % 2)
        plsc.subcore_barrier()

        def dump_body(j, _):
            pltpu.sync_copy(dacc.at[iota_v.at[j]], dump_v.at[j])
            return 0

        lax.fori_loop(0, wrpt, dump_body, 0)
        pltpu.sync_copy(dump_v, out_hbm.at[c, s])

    return deg_kernel


def _make_scat_kernel(tot_w, n_pad):
    wpw = tot_w // (NC * NS)
    chunks = wpw // CWS
    assert CWS * chunks == wpw and chunks >= 2
    rpt = n_pad // NS

    @functools.partial(
        pl.kernel,
        mesh=_sc_mesh(),
        compiler_params=pltpu.CompilerParams(use_tc_tiling_on_sc=False),
        out_type=jax.ShapeDtypeStruct((NC, n_pad, 16), jnp.float32),
        scratch_types=[
            pltpu.VMEM((4, CWS, LW), jnp.int32),
            pltpu.VMEM((4, CWS, LW), jnp.int32),
            pltpu.VMEM((2, CWS, LW, 16), jnp.float32),
            pltpu.VMEM_SHARED((n_pad, 16), jnp.float32),
            pltpu.SemaphoreType.DMA((2,)),
            pltpu.SemaphoreType.DMA((2,)),
            pltpu.SemaphoreType.DMA,
        ],
    )
    def scat_kernel(src_hbm, dst_hbm, u_hbm, zeros16_hbm, out_hbm,
                    sbuf, dbuf, gbuf, acc, gsem, ssem, isem):
        c = lax.axis_index("c")
        s = lax.axis_index("s")
        row0 = (c * NS + s) * wpw
        u_src = u_hbm
        pltpu.sync_copy(zeros16_hbm.at[pl.ds(s * rpt, rpt)],
                        acc.at[pl.ds(s * rpt, rpt)])
        plsc.subcore_barrier()

        def drain_gather(gsl, isl):
            def d(j, _):
                pltpu.make_async_copy(
                    u_src.at[sbuf.at[isl, j]], gbuf.at[gsl, j],
                    gsem.at[gsl]).wait()
                return 0
            lax.fori_loop(0, CWS, d, 0)

        def fire_scatter(gsl, isl):
            def f(j, _):
                pltpu.async_copy(gbuf.at[gsl, j], acc.at[dbuf.at[isl, j]],
                                 ssem.at[gsl], add=True)
                return 0
            lax.fori_loop(0, CWS, f, 0)

        def drain_scatter(gsl, isl):
            def d(j, _):
                pltpu.make_async_copy(
                    gbuf.at[gsl, j], acc.at[dbuf.at[isl, j]],
                    ssem.at[gsl]).wait()
                return 0
            lax.fori_loop(0, CWS, d, 0)

        pltpu.sync_copy(src_hbm.at[pl.ds(row0, CWS)], sbuf.at[0])
        pltpu.sync_copy(dst_hbm.at[pl.ds(row0, CWS)], dbuf.at[0])

        def chunk_body(ch, _):
            ia = lax.rem(ch, 4)
            ip = lax.rem(ch + 1, 4)
            ga = jnp.bitwise_and(ch, 1)
            gb = 1 - ga

            @pl.when(ch >= 2)
            def _():
                drain_scatter(ga, lax.rem(ch - 2, 4))

            @pl.when(ch + 1 < chunks)
            def _():
                pltpu.async_copy(src_hbm.at[pl.ds(row0 + (ch + 1) * CWS, CWS)],
                                 sbuf.at[ip], isem)
                pltpu.async_copy(dst_hbm.at[pl.ds(row0 + (ch + 1) * CWS, CWS)],
                                 dbuf.at[ip], isem)

            @pl.when(ch >= 1)
            def _():
                pltpu.make_async_copy(
                    src_hbm.at[pl.ds(row0 + ch * CWS, CWS)],
                    sbuf.at[ia], isem).wait()
                pltpu.make_async_copy(
                    dst_hbm.at[pl.ds(row0 + ch * CWS, CWS)],
                    dbuf.at[ia], isem).wait()

            def fire_gather(j, _):
                pltpu.async_copy(u_src.at[sbuf.at[ia, j]], gbuf.at[ga, j],
                                 gsem.at[ga])
                return 0

            lax.fori_loop(0, CWS, fire_gather, 0)

            @pl.when(ch >= 1)
            def _():
                drain_gather(gb, lax.rem(ch - 1, 4))
                fire_scatter(gb, lax.rem(ch - 1, 4))

            return 0

        lax.fori_loop(0, chunks, chunk_body, 0)
        lastg = (chunks - 1) % 2
        lasti = (chunks - 1) % 4
        drain_gather(lastg, lasti)
        fire_scatter(lastg, lasti)
        drain_scatter((chunks - 2) % 2, (chunks - 2) % 4)
        drain_scatter(lastg, lasti)
        plsc.subcore_barrier()
        pltpu.sync_copy(acc.at[pl.ds(s * rpt, rpt)],
                        out_hbm.at[c, pl.ds(s * rpt, rpt)])

    return scat_kernel


def _k1_body(deg_ref, x_ref, w1_ref, dinv_ref, u1_ref):
    deg = deg_ref[0] + deg_ref[1] + 1.0
    dv = jnp.transpose(lax.rsqrt(deg))
    pieces = [jnp.broadcast_to(dv[:, p:p + 1], (128, 16))
              for p in range(NB // 128)]
    dinv16 = jnp.concatenate(pieces, axis=0)
    x = x_ref[...]
    w1 = w1_ref[...]
    h = (x[:, 0:1] * w1[0:1, :] + x[:, 1:2] * w1[1:2, :]
         + x[:, 2:3] * w1[2:3, :])
    dinv_ref[...] = dinv16
    u1_ref[...] = dinv16 * h


def _call_k1(n, n_pad, degp, x, w1):
    grid = (n + NB - 1) // NB
    return pl.pallas_call(
        _k1_body,
        grid=(grid,),
        in_specs=[
            pl.BlockSpec((NC, NB // 128, 128), lambda i: (0, i, 0)),
            pl.BlockSpec((NB, 3), lambda i: (i, 0)),
            pl.BlockSpec((3, 16), lambda i: (0, 0)),
        ],
        out_specs=[
            pl.BlockSpec((NB, 16), lambda i: (i, 0)),
            pl.BlockSpec((NB, 16), lambda i: (i, 0)),
        ],
        out_shape=[
            jax.ShapeDtypeStruct((n, 16), jnp.float32),
            jax.ShapeDtypeStruct((n, 16), jnp.float32),
        ],
    )(degp.reshape(NC, n_pad // 128, 128), x, w1)


def _k3_body(s1_ref, u1_ref, dinv_ref, b1_ref, w2_ref, u2a_ref, u2b_ref):
    dinv = dinv_ref[...]
    h = dinv * (s1_ref[0] + s1_ref[1] + u1_ref[...]) + b1_ref[...]
    h = jnp.maximum(h, 0.0)
    t = jnp.dot(h, w2_ref[...], preferred_element_type=jnp.float32)
    u2 = jnp.concatenate([dinv, dinv], axis=1) * t
    u2a_ref[...] = u2[:, :16]
    u2b_ref[...] = u2[:, 16:]


def _call_k3(n, s1, u1, dinv, b1, w2):
    grid = (n + NB - 1) // NB
    return pl.pallas_call(
        _k3_body,
        grid=(grid,),
        in_specs=[
            pl.BlockSpec((NC, NB, 16), lambda i: (0, i, 0)),
            pl.BlockSpec((NB, 16), lambda i: (i, 0)),
            pl.BlockSpec((NB, 16), lambda i: (i, 0)),
            pl.BlockSpec((1, 16), lambda i: (0, 0)),
            pl.BlockSpec((16, 32), lambda i: (0, 0)),
        ],
        out_specs=[
            pl.BlockSpec((NB, 16), lambda i: (i, 0)),
            pl.BlockSpec((NB, 16), lambda i: (i, 0)),
        ],
        out_shape=[
            jax.ShapeDtypeStruct((n, 16), jnp.float32),
            jax.ShapeDtypeStruct((n, 16), jnp.float32),
        ],
    )(s1, u1, dinv, b1, w2)


def _k5_body(n, num_g, grid, s2a_ref, s2b_ref, u2a_ref, u2b_ref, dinv_ref,
             b2_ref, batch_ref, wlin_ref, blin_ref, out_ref, acc, cnt):
    i = pl.program_id(0)
    dinv = dinv_ref[...]
    ha = s2a_ref[0] + s2a_ref[1] + u2a_ref[...]
    hb = s2b_ref[0] + s2b_ref[1] + u2b_ref[...]
    h = (jnp.concatenate([dinv * ha, dinv * hb], axis=1)
         + b2_ref[...])
    h = jnp.maximum(h, 0.0)
    row = i * NB + lax.broadcasted_iota(jnp.int32, (NB, 1), 0)
    valid = row < n
    g0 = (batch_ref[0, 0] // 8) * 8
    cols = g0 + lax.broadcasted_iota(jnp.int32, (NB, GW), 1)
    onehot = jnp.where((batch_ref[...] == cols) & valid, 1.0, 0.0)

    @pl.when(i == 0)
    def _():
        acc[...] = jnp.zeros_like(acc)
        cnt[...] = jnp.zeros_like(cnt)

    acc[pl.ds(g0, GW), :] += lax.dot_general(
        onehot, h, (((0,), (0,)), ((), ())),
        preferred_element_type=jnp.float32)
    cnt[pl.ds(g0, GW), :] += lax.dot_general(
        onehot, jnp.ones((NB, 1), jnp.float32), (((0,), (0,)), ((), ())),
        preferred_element_type=jnp.float32)

    @pl.when(i == grid - 1)
    def _():
        g = acc[pl.ds(0, num_g), :] / jnp.maximum(cnt[pl.ds(0, num_g), :], 1.0)
        out_ref[...] = jnp.dot(g, wlin_ref[...],
                               preferred_element_type=jnp.float32) + blin_ref[...]


def _call_k5(n, num_g, s2a, s2b, u2a, u2b, dinv, b2, batch, wlin, blin):
    grid = (n + NB - 1) // NB
    body = functools.partial(_k5_body, n, num_g, grid)
    return pl.pallas_call(
        body,
        grid=(grid,),
        in_specs=[
            pl.BlockSpec((NC, NB, 16), lambda i: (0, i, 0)),
            pl.BlockSpec((NC, NB, 16), lambda i: (0, i, 0)),
            pl.BlockSpec((NB, 16), lambda i: (i, 0)),
            pl.BlockSpec((NB, 16), lambda i: (i, 0)),
            pl.BlockSpec((NB, 16), lambda i: (i, 0)),
            pl.BlockSpec((1, 32), lambda i: (0, 0)),
            pl.BlockSpec((NB, 1), lambda i: (i, 0)),
            pl.BlockSpec((32, 3), lambda i: (0, 0)),
            pl.BlockSpec((1, 3), lambda i: (0, 0)),
        ],
        out_specs=pl.BlockSpec((num_g, 3), lambda i: (0, 0)),
        out_shape=jax.ShapeDtypeStruct((num_g, 3), jnp.float32),
        scratch_shapes=[
            pltpu.VMEM((num_g + GW, 32), jnp.float32),
            pltpu.VMEM((num_g + GW, 1), jnp.float32),
        ],
    )(s2a, s2b, u2a, u2b, dinv, b2, batch, wlin, blin)


def kernel(x, edge_index, batch, W1, b1, W2, b2, Wlin, blin):
    n = x.shape[0]
    e = edge_index.shape[1]
    num_g = 512

    unit = LW * NC * NS * CW
    e_pad = ((e + unit - 1) // unit) * unit
    tot_w = e_pad // LW
    n_pad = ((n + NS * LW) // (NS * LW)) * (NS * LW)

    src = edge_index[0]
    dst = edge_index[1]
    pad = e_pad - e
    srcp = jnp.concatenate(
        [src, jnp.zeros((pad,), jnp.int32)]).reshape(tot_w, LW)
    dstp = jnp.concatenate(
        [dst, jnp.full((pad,), n, jnp.int32)]).reshape(tot_w, LW)
    zeros16 = jnp.zeros((n_pad, 16), jnp.float32)
    ones128 = jnp.ones((LW,), jnp.float32)
    zeros128 = jnp.zeros((LW,), jnp.float32)
    iota_nm = jnp.arange(n_pad, dtype=jnp.int32).reshape(
        NS, n_pad // NS // LW, LW)

    deg_k = _make_deg_kernel(tot_w, n_pad)
    scat_k = _make_scat_kernel(tot_w, n_pad)

    degp = deg_k(dstp, iota_nm, ones128, zeros128)
    dinv, u1 = _call_k1(n, n_pad, degp, x, W1)
    s1 = scat_k(srcp, dstp, u1, zeros16)
    u2a, u2b = _call_k3(n, s1, u1, dinv, b1.reshape(1, 16), W2)
    s2a = scat_k(srcp, dstp, u2a, zeros16)
    s2b = scat_k(srcp, dstp, u2b, zeros16)
    return _call_k5(n, num_g, s2a, s2b, u2a, u2b, dinv,
                    b2.reshape(1, 32), batch.reshape(n, 1), Wlin,
                    blin.reshape(1, 3))


# speedup vs baseline: 61.9836x; 1.0044x over previous
import functools

import jax
import jax.numpy as jnp
from jax import lax
from jax.experimental import pallas as pl
from jax.experimental.pallas import tpu as pltpu
from jax.experimental.pallas import tpu_sc as plsc

NC = 2
NS = 16
LW = 128
CW = 16
CWS = 4

NB = 4096
NBL = 8192
GW = 72


def _sc_mesh():
    return plsc.VectorSubcoreMesh(core_axis_name="c", subcore_axis_name="s")


def _make_deg_kernel(tot_w, n_pad):
    wpw = tot_w // (NC * NS)
    chunks = wpw // CW
    assert CW * chunks == wpw and chunks >= 2
    rpt = n_pad // NS
    wrpt = rpt // LW
    assert wrpt * LW == rpt

    @functools.partial(
        pl.kernel,
        mesh=_sc_mesh(),
        out_type=jax.ShapeDtypeStruct((NC, NS, wrpt, LW), jnp.float32),
        scratch_types=[
            pltpu.VMEM((2, CW, LW), jnp.int32),
            pltpu.VMEM((LW,), jnp.float32),
            pltpu.VMEM((LW,), jnp.float32),
            pltpu.VMEM((wrpt, LW), jnp.int32),
            pltpu.VMEM((wrpt, LW), jnp.float32),
            pltpu.VMEM_SHARED((n_pad,), jnp.float32),
            pltpu.SemaphoreType.DMA((2,)),
        ],
    )
    def deg_kernel(dst_hbm, iota_hbm, ones_hbm, zeros_hbm, out_hbm,
                   dbuf, ones_v, zeros_v, iota_v, dump_v, dacc, ssem):
        c = lax.axis_index("c")
        s = lax.axis_index("s")
        wid = c * NS + s
        row0 = wid * wpw
        pltpu.sync_copy(ones_hbm, ones_v)
        pltpu.sync_copy(zeros_hbm, zeros_v)
        pltpu.sync_copy(iota_hbm.at[s], iota_v)

        def zero_body(j, _):
            pltpu.sync_copy(zeros_v, dacc.at[iota_v.at[j]])
            return 0

        lax.fori_loop(0, wrpt, zero_body, 0)
        plsc.subcore_barrier()

        def drain_slot(sl):
            def d(j, _):
                pltpu.make_async_copy(
                    ones_v, dacc.at[dbuf.at[sl, j]], ssem.at[sl]).wait()
                return 0
            lax.fori_loop(0, CW, d, 0)

        def chunk_body(ch, _):
            a = jnp.bitwise_and(ch, 1)

            @pl.when(ch >= 2)
            def _():
                drain_slot(a)

            pltpu.sync_copy(dst_hbm.at[pl.ds(row0 + ch * CW, CW)], dbuf.at[a])

            def fire(j, _):
                pltpu.async_copy(ones_v, dacc.at[dbuf.at[a, j]], ssem.at[a],
                                 add=True)
                return 0

            lax.fori_loop(0, CW, fire, 0)
            return 0

        lax.fori_loop(0, chunks, chunk_body, 0)
        drain_slot((chunks - 1) % 2)
        drain_slot((chunks - 2) % 2)
        plsc.subcore_barrier()

        def dump_body(j, _):
            pltpu.sync_copy(dacc.at[iota_v.at[j]], dump_v.at[j])
            return 0

        lax.fori_loop(0, wrpt, dump_body, 0)
        pltpu.sync_copy(dump_v, out_hbm.at[c, s])

    return deg_kernel


def _make_scat_kernel(tot_w, n_pad):
    wpw = tot_w // (NC * NS)
    chunks = wpw // CWS
    assert CWS * chunks == wpw and chunks >= 2
    rpt = n_pad // NS

    @functools.partial(
        pl.kernel,
        mesh=_sc_mesh(),
        compiler_params=pltpu.CompilerParams(use_tc_tiling_on_sc=False),
        out_type=jax.ShapeDtypeStruct((NC, n_pad, 16), jnp.float32),
        scratch_types=[
            pltpu.VMEM((4, CWS, LW), jnp.int32),
            pltpu.VMEM((4, CWS, LW), jnp.int32),
            pltpu.VMEM((2, CWS, LW, 16), jnp.float32),
            pltpu.VMEM_SHARED((n_pad, 16), jnp.float32),
            pltpu.SemaphoreType.DMA((2,)),
            pltpu.SemaphoreType.DMA((2,)),
            pltpu.SemaphoreType.DMA,
        ],
    )
    def scat_kernel(src_hbm, dst_hbm, u_hbm, zeros16_hbm, out_hbm,
                    sbuf, dbuf, gbuf, acc, gsem, ssem, isem):
        c = lax.axis_index("c")
        s = lax.axis_index("s")
        row0 = (c * NS + s) * wpw
        u_src = u_hbm
        pltpu.sync_copy(zeros16_hbm.at[pl.ds(s * rpt, rpt)],
                        acc.at[pl.ds(s * rpt, rpt)])
        plsc.subcore_barrier()

        def drain_gather(gsl, isl):
            def d(j, _):
                pltpu.make_async_copy(
                    u_src.at[sbuf.at[isl, j]], gbuf.at[gsl, j],
                    gsem.at[gsl]).wait()
                return 0
            lax.fori_loop(0, CWS, d, 0)

        def fire_scatter(gsl, isl):
            def f(j, _):
                pltpu.async_copy(gbuf.at[gsl, j], acc.at[dbuf.at[isl, j]],
                                 ssem.at[gsl], add=True)
                return 0
            lax.fori_loop(0, CWS, f, 0)

        def drain_scatter(gsl, isl):
            def d(j, _):
                pltpu.make_async_copy(
                    gbuf.at[gsl, j], acc.at[dbuf.at[isl, j]],
                    ssem.at[gsl]).wait()
                return 0
            lax.fori_loop(0, CWS, d, 0)

        pltpu.sync_copy(src_hbm.at[pl.ds(row0, CWS)], sbuf.at[0])
        pltpu.sync_copy(dst_hbm.at[pl.ds(row0, CWS)], dbuf.at[0])

        def chunk_body(ch, _):
            ia = lax.rem(ch, 4)
            ip = lax.rem(ch + 1, 4)
            ga = jnp.bitwise_and(ch, 1)
            gb = 1 - ga

            @pl.when(ch >= 2)
            def _():
                drain_scatter(ga, lax.rem(ch - 2, 4))

            @pl.when(ch + 1 < chunks)
            def _():
                pltpu.async_copy(src_hbm.at[pl.ds(row0 + (ch + 1) * CWS, CWS)],
                                 sbuf.at[ip], isem)
                pltpu.async_copy(dst_hbm.at[pl.ds(row0 + (ch + 1) * CWS, CWS)],
                                 dbuf.at[ip], isem)

            @pl.when(ch >= 1)
            def _():
                pltpu.make_async_copy(
                    src_hbm.at[pl.ds(row0 + ch * CWS, CWS)],
                    sbuf.at[ia], isem).wait()
                pltpu.make_async_copy(
                    dst_hbm.at[pl.ds(row0 + ch * CWS, CWS)],
                    dbuf.at[ia], isem).wait()

            def fire_gather(j, _):
                pltpu.async_copy(u_src.at[sbuf.at[ia, j]], gbuf.at[ga, j],
                                 gsem.at[ga])
                return 0

            lax.fori_loop(0, CWS, fire_gather, 0)

            @pl.when(ch >= 1)
            def _():
                drain_gather(gb, lax.rem(ch - 1, 4))
                fire_scatter(gb, lax.rem(ch - 1, 4))

            return 0

        lax.fori_loop(0, chunks, chunk_body, 0)
        lastg = (chunks - 1) % 2
        lasti = (chunks - 1) % 4
        drain_gather(lastg, lasti)
        fire_scatter(lastg, lasti)
        drain_scatter((chunks - 2) % 2, (chunks - 2) % 4)
        drain_scatter(lastg, lasti)
        plsc.subcore_barrier()
        pltpu.sync_copy(acc.at[pl.ds(s * rpt, rpt)],
                        out_hbm.at[c, pl.ds(s * rpt, rpt)])

    return scat_kernel


def _k1_body(deg_ref, x_ref, w1_ref, dinv_ref, u1_ref):
    deg = deg_ref[0] + deg_ref[1] + 1.0
    dv = jnp.transpose(lax.rsqrt(deg))
    pieces = [jnp.broadcast_to(dv[:, p:p + 1], (128, 16))
              for p in range(NBL // 128)]
    dinv16 = jnp.concatenate(pieces, axis=0)
    x = x_ref[...]
    w1 = w1_ref[...]
    h = (x[:, 0:1] * w1[0:1, :] + x[:, 1:2] * w1[1:2, :]
         + x[:, 2:3] * w1[2:3, :])
    dinv_ref[...] = dinv16
    u1_ref[...] = dinv16 * h


def _call_k1(n, n_pad, degp, x, w1):
    grid = (n + NBL - 1) // NBL
    return pl.pallas_call(
        _k1_body,
        grid=(grid,),
        in_specs=[
            pl.BlockSpec((NC, NBL // 128, 128), lambda i: (0, i, 0)),
            pl.BlockSpec((NBL, 3), lambda i: (i, 0)),
            pl.BlockSpec((3, 16), lambda i: (0, 0)),
        ],
        out_specs=[
            pl.BlockSpec((NBL, 16), lambda i: (i, 0)),
            pl.BlockSpec((NBL, 16), lambda i: (i, 0)),
        ],
        out_shape=[
            jax.ShapeDtypeStruct((n, 16), jnp.float32),
            jax.ShapeDtypeStruct((n, 16), jnp.float32),
        ],
    )(degp.reshape(NC, n_pad // 128, 128), x, w1)


def _k3_body(s1_ref, u1_ref, dinv_ref, b1_ref, w2_ref, u2a_ref, u2b_ref):
    dinv = dinv_ref[...]
    h = dinv * (s1_ref[0] + s1_ref[1] + u1_ref[...]) + b1_ref[...]
    h = jnp.maximum(h, 0.0)
    t = jnp.dot(h, w2_ref[...], preferred_element_type=jnp.float32)
    u2 = jnp.concatenate([dinv, dinv], axis=1) * t
    u2a_ref[...] = u2[:, :16]
    u2b_ref[...] = u2[:, 16:]


def _call_k3(n, s1, u1, dinv, b1, w2):
    grid = (n + NBL - 1) // NBL
    return pl.pallas_call(
        _k3_body,
        grid=(grid,),
        in_specs=[
            pl.BlockSpec((NC, NBL, 16), lambda i: (0, i, 0)),
            pl.BlockSpec((NBL, 16), lambda i: (i, 0)),
            pl.BlockSpec((NBL, 16), lambda i: (i, 0)),
            pl.BlockSpec((1, 16), lambda i: (0, 0)),
            pl.BlockSpec((16, 32), lambda i: (0, 0)),
        ],
        out_specs=[
            pl.BlockSpec((NBL, 16), lambda i: (i, 0)),
            pl.BlockSpec((NBL, 16), lambda i: (i, 0)),
        ],
        out_shape=[
            jax.ShapeDtypeStruct((n, 16), jnp.float32),
            jax.ShapeDtypeStruct((n, 16), jnp.float32),
        ],
    )(s1, u1, dinv, b1, w2)


def _k5_body(n, num_g, grid, s2a_ref, s2b_ref, u2a_ref, u2b_ref, dinv_ref,
             b2_ref, batch_ref, wlin_ref, blin_ref, out_ref, acc, cnt):
    i = pl.program_id(0)
    dinv = dinv_ref[...]
    ha = s2a_ref[0] + s2a_ref[1] + u2a_ref[...]
    hb = s2b_ref[0] + s2b_ref[1] + u2b_ref[...]
    h = (jnp.concatenate([dinv * ha, dinv * hb], axis=1)
         + b2_ref[...])
    h = jnp.maximum(h, 0.0)
    row = i * NB + lax.broadcasted_iota(jnp.int32, (NB, 1), 0)
    valid = row < n
    g0 = (batch_ref[0, 0] // 8) * 8
    cols = g0 + lax.broadcasted_iota(jnp.int32, (NB, GW), 1)
    onehot = jnp.where((batch_ref[...] == cols) & valid, 1.0, 0.0)

    @pl.when(i == 0)
    def _():
        acc[...] = jnp.zeros_like(acc)
        cnt[...] = jnp.zeros_like(cnt)

    acc[pl.ds(g0, GW), :] += lax.dot_general(
        onehot, h, (((0,), (0,)), ((), ())),
        preferred_element_type=jnp.float32)
    cnt[pl.ds(g0, GW), :] += lax.dot_general(
        onehot, jnp.ones((NB, 1), jnp.float32), (((0,), (0,)), ((), ())),
        preferred_element_type=jnp.float32)

    @pl.when(i == grid - 1)
    def _():
        g = acc[pl.ds(0, num_g), :] / jnp.maximum(cnt[pl.ds(0, num_g), :], 1.0)
        out_ref[...] = jnp.dot(g, wlin_ref[...],
                               preferred_element_type=jnp.float32) + blin_ref[...]


def _call_k5(n, num_g, s2a, s2b, u2a, u2b, dinv, b2, batch, wlin, blin):
    grid = (n + NB - 1) // NB
    body = functools.partial(_k5_body, n, num_g, grid)
    return pl.pallas_call(
        body,
        grid=(grid,),
        in_specs=[
            pl.BlockSpec((NC, NB, 16), lambda i: (0, i, 0)),
            pl.BlockSpec((NC, NB, 16), lambda i: (0, i, 0)),
            pl.BlockSpec((NB, 16), lambda i: (i, 0)),
            pl.BlockSpec((NB, 16), lambda i: (i, 0)),
            pl.BlockSpec((NB, 16), lambda i: (i, 0)),
            pl.BlockSpec((1, 32), lambda i: (0, 0)),
            pl.BlockSpec((NB, 1), lambda i: (i, 0)),
            pl.BlockSpec((32, 3), lambda i: (0, 0)),
            pl.BlockSpec((1, 3), lambda i: (0, 0)),
        ],
        out_specs=pl.BlockSpec((num_g, 3), lambda i: (0, 0)),
        out_shape=jax.ShapeDtypeStruct((num_g, 3), jnp.float32),
        scratch_shapes=[
            pltpu.VMEM((num_g + GW, 32), jnp.float32),
            pltpu.VMEM((num_g + GW, 1), jnp.float32),
        ],
    )(s2a, s2b, u2a, u2b, dinv, b2, batch, wlin, blin)


def kernel(x, edge_index, batch, W1, b1, W2, b2, Wlin, blin):
    n = x.shape[0]
    e = edge_index.shape[1]
    num_g = 512

    unit = LW * NC * NS * CW
    e_pad = ((e + unit - 1) // unit) * unit
    tot_w = e_pad // LW
    n_pad = ((n + NS * LW) // (NS * LW)) * (NS * LW)

    src = edge_index[0]
    dst = edge_index[1]
    pad = e_pad - e
    srcp = jnp.concatenate(
        [src, jnp.zeros((pad,), jnp.int32)]).reshape(tot_w, LW)
    dstp = jnp.concatenate(
        [dst, jnp.full((pad,), n, jnp.int32)]).reshape(tot_w, LW)
    zeros16 = jnp.zeros((n_pad, 16), jnp.float32)
    ones128 = jnp.ones((LW,), jnp.float32)
    zeros128 = jnp.zeros((LW,), jnp.float32)
    iota_nm = jnp.arange(n_pad, dtype=jnp.int32).reshape(
        NS, n_pad // NS // LW, LW)

    deg_k = _make_deg_kernel(tot_w, n_pad)
    scat_k = _make_scat_kernel(tot_w, n_pad)

    degp = deg_k(dstp, iota_nm, ones128, zeros128)
    dinv, u1 = _call_k1(n, n_pad, degp, x, W1)
    s1 = scat_k(srcp, dstp, u1, zeros16)
    u2a, u2b = _call_k3(n, s1, u1, dinv, b1.reshape(1, 16), W2)
    s2a = scat_k(srcp, dstp, u2a, zeros16)
    s2b = scat_k(srcp, dstp, u2b, zeros16)
    return _call_k5(n, num_g, s2a, s2b, u2a, u2b, dinv,
                    b2.reshape(1, 32), batch.reshape(n, 1), Wlin,
                    blin.reshape(1, 3))


# speedup vs baseline: 64.8611x; 1.0464x over previous
import functools

import jax
import jax.numpy as jnp
from jax import lax
from jax.experimental import pallas as pl
from jax.experimental.pallas import tpu as pltpu
from jax.experimental.pallas import tpu_sc as plsc

NC = 2
NS = 16
LW = 128
CW = 16
CWS = 4

NB = 4096
NBL = 8192
GW = 72


def _sc_mesh():
    return plsc.VectorSubcoreMesh(core_axis_name="c", subcore_axis_name="s")


def _make_deg_kernel(tot_w, n_pad):
    wpw = tot_w // (NC * NS)
    chunks = wpw // CW
    assert CW * chunks == wpw and chunks >= 2
    rpt = n_pad // NS
    wrpt = rpt // LW
    assert wrpt * LW == rpt

    @functools.partial(
        pl.kernel,
        mesh=_sc_mesh(),
        out_type=jax.ShapeDtypeStruct((NC, NS, wrpt, LW), jnp.float32),
        scratch_types=[
            pltpu.VMEM((2, CW, LW), jnp.int32),
            pltpu.VMEM((LW,), jnp.float32),
            pltpu.VMEM((LW,), jnp.float32),
            pltpu.VMEM((wrpt, LW), jnp.int32),
            pltpu.VMEM((wrpt, LW), jnp.float32),
            pltpu.VMEM_SHARED((n_pad,), jnp.float32),
            pltpu.SemaphoreType.DMA((2,)),
        ],
    )
    def deg_kernel(dst_hbm, iota_hbm, ones_hbm, zeros_hbm, out_hbm,
                   dbuf, ones_v, zeros_v, iota_v, dump_v, dacc, ssem):
        c = lax.axis_index("c")
        s = lax.axis_index("s")
        wid = c * NS + s
        row0 = wid * wpw
        pltpu.sync_copy(ones_hbm, ones_v)
        pltpu.sync_copy(zeros_hbm, zeros_v)
        pltpu.sync_copy(iota_hbm.at[s], iota_v)

        def zero_body(j, _):
            pltpu.sync_copy(zeros_v, dacc.at[iota_v.at[j]])
            return 0

        lax.fori_loop(0, wrpt, zero_body, 0)
        plsc.subcore_barrier()

        def drain_slot(sl):
            def d(j, _):
                pltpu.make_async_copy(
                    ones_v, dacc.at[dbuf.at[sl, j]], ssem.at[sl]).wait()
                return 0
            lax.fori_loop(0, CW, d, 0)

        def chunk_body(ch, _):
            a = jnp.bitwise_and(ch, 1)

            @pl.when(ch >= 2)
            def _():
                drain_slot(a)

            pltpu.sync_copy(dst_hbm.at[pl.ds(row0 + ch * CW, CW)], dbuf.at[a])

            def fire(j, _):
                pltpu.async_copy(ones_v, dacc.at[dbuf.at[a, j]], ssem.at[a],
                                 add=True)
                return 0

            lax.fori_loop(0, CW, fire, 0)
            return 0

        lax.fori_loop(0, chunks, chunk_body, 0)
        drain_slot((chunks - 1) % 2)
        drain_slot((chunks - 2) % 2)
        plsc.subcore_barrier()

        def dump_body(j, _):
            pltpu.sync_copy(dacc.at[iota_v.at[j]], dump_v.at[j])
            return 0

        lax.fori_loop(0, wrpt, dump_body, 0)
        pltpu.sync_copy(dump_v, out_hbm.at[c, s])

    return deg_kernel


def _make_scat_kernel(tot_w, n_pad):
    wpw = tot_w // (NC * NS)
    chunks = wpw // CWS
    assert CWS * chunks == wpw and chunks >= 2
    rpt = n_pad // NS

    @functools.partial(
        pl.kernel,
        mesh=_sc_mesh(),
        compiler_params=pltpu.CompilerParams(use_tc_tiling_on_sc=False),
        out_type=jax.ShapeDtypeStruct((NC, n_pad, 16), jnp.float32),
        scratch_types=[
            pltpu.VMEM((4, CWS, LW), jnp.int32),
            pltpu.VMEM((4, CWS, LW), jnp.int32),
            pltpu.VMEM((3, CWS, LW, 16), jnp.float32),
            pltpu.VMEM_SHARED((n_pad, 16), jnp.float32),
            pltpu.SemaphoreType.DMA((3,)),
            pltpu.SemaphoreType.DMA((3,)),
            pltpu.SemaphoreType.DMA,
        ],
    )
    def scat_kernel(src_hbm, dst_hbm, u_hbm, zeros16_hbm, out_hbm,
                    sbuf, dbuf, gbuf, acc, gsem, ssem, isem):
        c = lax.axis_index("c")
        s = lax.axis_index("s")
        row0 = (c * NS + s) * wpw
        u_src = u_hbm
        pltpu.sync_copy(zeros16_hbm.at[pl.ds(s * rpt, rpt)],
                        acc.at[pl.ds(s * rpt, rpt)])
        plsc.subcore_barrier()

        def drain_gather(gsl, isl):
            def d(j, _):
                pltpu.make_async_copy(
                    u_src.at[sbuf.at[isl, j]], gbuf.at[gsl, j],
                    gsem.at[gsl]).wait()
                return 0
            lax.fori_loop(0, CWS, d, 0)

        def fire_scatter(gsl, isl):
            def f(j, _):
                pltpu.async_copy(gbuf.at[gsl, j], acc.at[dbuf.at[isl, j]],
                                 ssem.at[gsl], add=True)
                return 0
            lax.fori_loop(0, CWS, f, 0)

        def drain_scatter(gsl, isl):
            def d(j, _):
                pltpu.make_async_copy(
                    gbuf.at[gsl, j], acc.at[dbuf.at[isl, j]],
                    ssem.at[gsl]).wait()
                return 0
            lax.fori_loop(0, CWS, d, 0)

        pltpu.sync_copy(src_hbm.at[pl.ds(row0, CWS)], sbuf.at[0])
        pltpu.sync_copy(dst_hbm.at[pl.ds(row0, CWS)], dbuf.at[0])

        def chunk_body(ch, _):
            ia = lax.rem(ch, 4)
            ip = lax.rem(ch + 1, 4)
            ga = lax.rem(ch, 3)
            gprev = lax.rem(ch + 2, 3)

            @pl.when(ch >= 3)
            def _():
                drain_scatter(ga, lax.rem(ch - 3, 4))

            @pl.when(ch + 1 < chunks)
            def _():
                pltpu.async_copy(src_hbm.at[pl.ds(row0 + (ch + 1) * CWS, CWS)],
                                 sbuf.at[ip], isem)
                pltpu.async_copy(dst_hbm.at[pl.ds(row0 + (ch + 1) * CWS, CWS)],
                                 dbuf.at[ip], isem)

            @pl.when(ch >= 1)
            def _():
                pltpu.make_async_copy(
                    src_hbm.at[pl.ds(row0 + ch * CWS, CWS)],
                    sbuf.at[ia], isem).wait()
                pltpu.make_async_copy(
                    dst_hbm.at[pl.ds(row0 + ch * CWS, CWS)],
                    dbuf.at[ia], isem).wait()

            def fire_gather(j, _):
                pltpu.async_copy(u_src.at[sbuf.at[ia, j]], gbuf.at[ga, j],
                                 gsem.at[ga])
                return 0

            lax.fori_loop(0, CWS, fire_gather, 0)

            @pl.when(ch >= 1)
            def _():
                drain_gather(gprev, lax.rem(ch - 1, 4))
                fire_scatter(gprev, lax.rem(ch - 1, 4))

            return 0

        lax.fori_loop(0, chunks, chunk_body, 0)
        lastg = (chunks - 1) % 3
        lasti = (chunks - 1) % 4
        drain_gather(lastg, lasti)
        fire_scatter(lastg, lasti)
        drain_scatter((chunks - 3) % 3, (chunks - 3) % 4)
        drain_scatter((chunks - 2) % 3, (chunks - 2) % 4)
        drain_scatter(lastg, lasti)
        plsc.subcore_barrier()
        pltpu.sync_copy(acc.at[pl.ds(s * rpt, rpt)],
                        out_hbm.at[c, pl.ds(s * rpt, rpt)])

    return scat_kernel


def _k1_body(deg_ref, x_ref, w1_ref, dinv_ref, u1_ref):
    deg = deg_ref[0] + deg_ref[1] + 1.0
    dv = jnp.transpose(lax.rsqrt(deg))
    pieces = [jnp.broadcast_to(dv[:, p:p + 1], (128, 16))
              for p in range(NBL // 128)]
    dinv16 = jnp.concatenate(pieces, axis=0)
    x = x_ref[...]
    w1 = w1_ref[...]
    h = (x[:, 0:1] * w1[0:1, :] + x[:, 1:2] * w1[1:2, :]
         + x[:, 2:3] * w1[2:3, :])
    dinv_ref[...] = dinv16
    u1_ref[...] = dinv16 * h


def _call_k1(n, n_pad, degp, x, w1):
    grid = (n + NBL - 1) // NBL
    return pl.pallas_call(
        _k1_body,
        grid=(grid,),
        in_specs=[
            pl.BlockSpec((NC, NBL // 128, 128), lambda i: (0, i, 0)),
            pl.BlockSpec((NBL, 3), lambda i: (i, 0)),
            pl.BlockSpec((3, 16), lambda i: (0, 0)),
        ],
        out_specs=[
            pl.BlockSpec((NBL, 16), lambda i: (i, 0)),
            pl.BlockSpec((NBL, 16), lambda i: (i, 0)),
        ],
        out_shape=[
            jax.ShapeDtypeStruct((n, 16), jnp.float32),
            jax.ShapeDtypeStruct((n, 16), jnp.float32),
        ],
    )(degp.reshape(NC, n_pad // 128, 128), x, w1)


def _k3_body(s1_ref, u1_ref, dinv_ref, b1_ref, w2_ref, u2a_ref, u2b_ref):
    dinv = dinv_ref[...]
    h = dinv * (s1_ref[0] + s1_ref[1] + u1_ref[...]) + b1_ref[...]
    h = jnp.maximum(h, 0.0)
    t = jnp.dot(h, w2_ref[...], preferred_element_type=jnp.float32)
    u2 = jnp.concatenate([dinv, dinv], axis=1) * t
    u2a_ref[...] = u2[:, :16]
    u2b_ref[...] = u2[:, 16:]


def _call_k3(n, s1, u1, dinv, b1, w2):
    grid = (n + NBL - 1) // NBL
    return pl.pallas_call(
        _k3_body,
        grid=(grid,),
        in_specs=[
            pl.BlockSpec((NC, NBL, 16), lambda i: (0, i, 0)),
            pl.BlockSpec((NBL, 16), lambda i: (i, 0)),
            pl.BlockSpec((NBL, 16), lambda i: (i, 0)),
            pl.BlockSpec((1, 16), lambda i: (0, 0)),
            pl.BlockSpec((16, 32), lambda i: (0, 0)),
        ],
        out_specs=[
            pl.BlockSpec((NBL, 16), lambda i: (i, 0)),
            pl.BlockSpec((NBL, 16), lambda i: (i, 0)),
        ],
        out_shape=[
            jax.ShapeDtypeStruct((n, 16), jnp.float32),
            jax.ShapeDtypeStruct((n, 16), jnp.float32),
        ],
    )(s1, u1, dinv, b1, w2)


def _k5_body(n, num_g, grid, s2a_ref, s2b_ref, u2a_ref, u2b_ref, dinv_ref,
             b2_ref, batch_ref, wlin_ref, blin_ref, out_ref, acc, cnt):
    i = pl.program_id(0)
    dinv = dinv_ref[...]
    ha = s2a_ref[0] + s2a_ref[1] + u2a_ref[...]
    hb = s2b_ref[0] + s2b_ref[1] + u2b_ref[...]
    h = (jnp.concatenate([dinv * ha, dinv * hb], axis=1)
         + b2_ref[...])
    h = jnp.maximum(h, 0.0)
    row = i * NB + lax.broadcasted_iota(jnp.int32, (NB, 1), 0)
    valid = row < n
    g0 = (batch_ref[0, 0] // 8) * 8
    cols = g0 + lax.broadcasted_iota(jnp.int32, (NB, GW), 1)
    onehot = jnp.where((batch_ref[...] == cols) & valid, 1.0, 0.0)

    @pl.when(i == 0)
    def _():
        acc[...] = jnp.zeros_like(acc)
        cnt[...] = jnp.zeros_like(cnt)

    acc[pl.ds(g0, GW), :] += lax.dot_general(
        onehot, h, (((0,), (0,)), ((), ())),
        preferred_element_type=jnp.float32)
    cnt[pl.ds(g0, GW), :] += lax.dot_general(
        onehot, jnp.ones((NB, 1), jnp.float32), (((0,), (0,)), ((), ())),
        preferred_element_type=jnp.float32)

    @pl.when(i == grid - 1)
    def _():
        g = acc[pl.ds(0, num_g), :] / jnp.maximum(cnt[pl.ds(0, num_g), :], 1.0)
        out_ref[...] = jnp.dot(g, wlin_ref[...],
                               preferred_element_type=jnp.float32) + blin_ref[...]


def _call_k5(n, num_g, s2a, s2b, u2a, u2b, dinv, b2, batch, wlin, blin):
    grid = (n + NB - 1) // NB
    body = functools.partial(_k5_body, n, num_g, grid)
    return pl.pallas_call(
        body,
        grid=(grid,),
        in_specs=[
            pl.BlockSpec((NC, NB, 16), lambda i: (0, i, 0)),
            pl.BlockSpec((NC, NB, 16), lambda i: (0, i, 0)),
            pl.BlockSpec((NB, 16), lambda i: (i, 0)),
            pl.BlockSpec((NB, 16), lambda i: (i, 0)),
            pl.BlockSpec((NB, 16), lambda i: (i, 0)),
            pl.BlockSpec((1, 32), lambda i: (0, 0)),
            pl.BlockSpec((NB, 1), lambda i: (i, 0)),
            pl.BlockSpec((32, 3), lambda i: (0, 0)),
            pl.BlockSpec((1, 3), lambda i: (0, 0)),
        ],
        out_specs=pl.BlockSpec((num_g, 3), lambda i: (0, 0)),
        out_shape=jax.ShapeDtypeStruct((num_g, 3), jnp.float32),
        scratch_shapes=[
            pltpu.VMEM((num_g + GW, 32), jnp.float32),
            pltpu.VMEM((num_g + GW, 1), jnp.float32),
        ],
    )(s2a, s2b, u2a, u2b, dinv, b2, batch, wlin, blin)


def kernel(x, edge_index, batch, W1, b1, W2, b2, Wlin, blin):
    n = x.shape[0]
    e = edge_index.shape[1]
    num_g = 512

    unit = LW * NC * NS * CW
    e_pad = ((e + unit - 1) // unit) * unit
    tot_w = e_pad // LW
    n_pad = ((n + NS * LW) // (NS * LW)) * (NS * LW)

    src = edge_index[0]
    dst = edge_index[1]
    pad = e_pad - e
    srcp = jnp.concatenate(
        [src, jnp.zeros((pad,), jnp.int32)]).reshape(tot_w, LW)
    dstp = jnp.concatenate(
        [dst, jnp.full((pad,), n, jnp.int32)]).reshape(tot_w, LW)
    zeros16 = jnp.zeros((n_pad, 16), jnp.float32)
    ones128 = jnp.ones((LW,), jnp.float32)
    zeros128 = jnp.zeros((LW,), jnp.float32)
    iota_nm = jnp.arange(n_pad, dtype=jnp.int32).reshape(
        NS, n_pad // NS // LW, LW)

    deg_k = _make_deg_kernel(tot_w, n_pad)
    scat_k = _make_scat_kernel(tot_w, n_pad)

    degp = deg_k(dstp, iota_nm, ones128, zeros128)
    dinv, u1 = _call_k1(n, n_pad, degp, x, W1)
    s1 = scat_k(srcp, dstp, u1, zeros16)
    u2a, u2b = _call_k3(n, s1, u1, dinv, b1.reshape(1, 16), W2)
    s2a = scat_k(srcp, dstp, u2a, zeros16)
    s2b = scat_k(srcp, dstp, u2b, zeros16)
    return _call_k5(n, num_g, s2a, s2b, u2a, u2b, dinv,
                    b2.reshape(1, 32), batch.reshape(n, 1), Wlin,
                    blin.reshape(1, 3))
